# Initial kernel scaffold; baseline (speedup 1.0000x reference)
import functools

import jax
import jax.numpy as jnp
from jax import lax
from jax.experimental import pallas as pl
from jax.experimental.pallas import tpu as pltpu
from jax.experimental.pallas import tpu_sc as plsc

N = 10000
E = 320000
C = 128
R = 8
CW = 144
P = 1680
P8 = P * 8
NRANGE = 6
NPASS = 3
NSUB = 16
EPT = E // NSUB
CAP = 20480
B = 128
ACC_ROWS = 13568
DUMMY = P8
ZROWS = 32


def _sc_body(xa, keyh, srch, outh,
             key_v, src_v, cidx, csrc, idxs, srcs, rows, zbuf, acc, sem):
    c = lax.axis_index("c")
    s = lax.axis_index("s")

    pltpu.sync_copy(keyh.at[pl.ds(s * EPT, EPT)], key_v)
    pltpu.sync_copy(srch.at[pl.ds(s * EPT, EPT)], src_v)

    zero16 = jnp.zeros((16,), jnp.float32)
    for zr in range(ZROWS):
        for zc in range(CW // 16):
            zbuf[zr, pl.ds(zc * 16, 16)] = zero16

    iota = lax.iota(jnp.int32, 16)
    dummyv = jnp.full((16,), DUMMY, jnp.int32)
    zerov = jnp.zeros((16,), jnp.int32)

    for pass_ in range(NPASS):
        rid = pass_ * 2 + c
        lo8 = rid * P8

        zbase = s * (ACC_ROWS // NSUB)
        def z_body(j, _):
            pltpu.sync_copy(zbuf, acc.at[pl.ds(zbase + j * ZROWS, ZROWS)])
            return 0
        lax.fori_loop(0, 26, z_body, 0)
        pltpu.sync_copy(zbuf.at[pl.ds(0, 16)], acc.at[pl.ds(zbase + 832, 16)])
        plsc.subcore_barrier()

        lo8v = jnp.full((16,), lo8, jnp.int32)
        def scan_body(i, off):
            k = key_v[pl.ds(i * 16, 16)]
            m = (k >= lo8v) & (k < lo8v + P8)
            pos = off + plsc.cumsum(m.astype(jnp.int32)) - 1
            plsc.store_scatter(cidx, [pos], k - lo8v, m)
            sv = src_v[pl.ds(i * 16, 16)]
            plsc.store_scatter(csrc, [pos], sv, m)
            return off + plsc.all_reduce_population_count(m)
        off_v = lax.fori_loop(0, EPT // 16, scan_body,
                              jnp.zeros((16,), jnp.int32))
        off = jnp.max(off_v)

        for j in range(B // 16):
            pos = off_v + (j * 16) + iota
            plsc.store_scatter(cidx, [pos], dummyv)
            plsc.store_scatter(csrc, [pos], zerov)

        nb = lax.shift_right_logical(off + (B - 1), 7)
        def g_body(j, _):
            pltpu.sync_copy(cidx.at[pl.ds(j * B, B)], idxs)
            pltpu.sync_copy(csrc.at[pl.ds(j * B, B)], srcs)
            pltpu.async_copy(xa.at[srcs], rows, sem).wait()
            pltpu.sync_copy(rows, acc.at[idxs], add=True)
            return 0
        lax.fori_loop(0, nb, g_body, 0)
        plsc.subcore_barrier()

        obase = s * (P8 // NSUB)
        def o_body(j, _):
            pltpu.sync_copy(acc.at[pl.ds(obase + j * 60, 60)],
                            outh.at[rid, pl.ds(obase + j * 60, 60)])
            return 0
        lax.fori_loop(0, 14, o_body, 0)
        plsc.subcore_barrier()


def _sc_scatter(xa, key, src):
    mesh = plsc.VectorSubcoreMesh(core_axis_name="c", subcore_axis_name="s")
    return pl.kernel(
        _sc_body,
        mesh=mesh,
        out_type=jax.ShapeDtypeStruct((NRANGE, P8, CW), jnp.float32),
        scratch_types=[
            pltpu.VMEM((EPT,), jnp.int32),
            pltpu.VMEM((EPT,), jnp.int32),
            pltpu.VMEM((CAP,), jnp.int32),
            pltpu.VMEM((CAP,), jnp.int32),
            pltpu.VMEM((B,), jnp.int32),
            pltpu.VMEM((B,), jnp.int32),
            pltpu.VMEM((B, CW), jnp.float32),
            pltpu.VMEM((ZROWS, CW), jnp.float32),
            pltpu.VMEM_SHARED((ACC_ROWS, CW), jnp.float32),
            pltpu.SemaphoreType.DMA,
        ],
    )(xa, key, src)


def _tc_body(x_ref, s_ref, w_ref, root_ref, bias_ref, o_ref):
    acc = jnp.dot(x_ref[...], root_ref[...],
                  preferred_element_type=jnp.float32) + bias_ref[...]
    for t in range(R):
        st = s_ref[:, t, :]
        cnt = jnp.maximum(st[:, 128:129], 1.0)
        mean = st[:, :128] / cnt
        acc = acc + jnp.dot(mean, w_ref[t], preferred_element_type=jnp.float32)
    o_ref[...] = jnp.maximum(acc, 0.0)


def _tc_combine(x, s3, weight, root, bias2):
    BN = 400
    grid = (N // BN,)
    return pl.pallas_call(
        _tc_body,
        grid=grid,
        in_specs=[
            pl.BlockSpec((BN, C), lambda i: (i, 0)),
            pl.BlockSpec((BN, R, CW), lambda i: (i, 0, 0)),
            pl.BlockSpec((R, C, C), lambda i: (0, 0, 0)),
            pl.BlockSpec((C, C), lambda i: (0, 0)),
            pl.BlockSpec((1, C), lambda i: (0, 0)),
        ],
        out_specs=pl.BlockSpec((BN, C), lambda i: (i, 0)),
        out_shape=jax.ShapeDtypeStruct((N, C), jnp.float32),
    )(x, s3, weight, root, bias2)


@jax.jit
def kernel(node_features, edge_index, edge_type, weight, root, bias):
    x = node_features.astype(jnp.float32)
    src = edge_index[0].astype(jnp.int32)
    dst = edge_index[1].astype(jnp.int32)
    et = edge_type.astype(jnp.int32)
    key = dst * 8 + et
    xa = jnp.concatenate(
        [x, jnp.ones((N, 1), jnp.float32), jnp.zeros((N, CW - C - 1), jnp.float32)],
        axis=1)
    s = _sc_scatter(xa, key, src)
    s3 = s.reshape(NRANGE * P, R, CW)
    return _tc_combine(x, s3, weight, root.astype(jnp.float32),
                       bias.reshape(1, C).astype(jnp.float32))


# speedup vs baseline: 9.1659x; 9.1659x over previous
import jax
import jax.numpy as jnp
from jax import lax
from jax.experimental import pallas as pl
from jax.experimental.pallas import tpu as pltpu
from jax.experimental.pallas import tpu_sc as plsc

N = 10000
E = 320000
C = 128
R = 8
P = 848
P8 = P * 8
NRANGE = 12
NPASS = 6
NSUB = 16
EPT = E // NSUB
ECH = 2000
NCH = EPT // ECH
CAP = 20480
B = 128
ACC_ROWS = 6912
DUMMY = P8
ZROWS = 32
CROWS = 56


def _sc_body(xa, keyh, srch, identh, outh, outch,
             key_c, src_c, cidx, csrc, rows, zbuf, cnt_v, ident, acc, cnt_sh,
             sem):
    c = lax.axis_index("c")
    s = lax.axis_index("s")

    pltpu.sync_copy(identh, ident)

    zero16f = jnp.zeros((16,), jnp.float32)
    for zr in range(ZROWS):
        for zc in range(C // 16):
            zbuf[zr, pl.ds(zc * 16, 16)] = zero16f

    iota = lax.iota(jnp.int32, 16)
    dummyv = jnp.full((16,), DUMMY, jnp.int32)
    zerov = jnp.zeros((16,), jnp.int32)
    ones16f = jnp.ones((16,), jnp.float32)

    for pass_ in range(NPASS):
        rid = pass_ * 2 + c
        lo8 = rid * P8

        zbase = s * (ACC_ROWS // NSUB)
        def z_body(j, _):
            pltpu.sync_copy(zbuf, acc.at[pl.ds(zbase + j * ZROWS, ZROWS)])
            return 0
        lax.fori_loop(0, 13, z_body, 0)
        pltpu.sync_copy(zbuf.at[pl.ds(0, 16)],
                        acc.at[pl.ds(zbase + 416, 16)])
        @pl.when(s < CROWS // 8)
        def _():
            pltpu.sync_copy(zbuf.at[pl.ds(0, 8)], cnt_sh.at[pl.ds(s * 8, 8)])
        def zc_body(j, _):
            for zc in range(C // 16):
                cnt_v[j, pl.ds(zc * 16, 16)] = zero16f
            return 0
        lax.fori_loop(0, CROWS, zc_body, 0)
        plsc.subcore_barrier()

        lo8v = jnp.full((16,), lo8, jnp.int32)
        def chunk_body(ch, off_c):
            ebase = s * EPT + ch * ECH
            pltpu.sync_copy(keyh.at[pl.ds(ebase, ECH)], key_c)
            pltpu.sync_copy(srch.at[pl.ds(ebase, ECH)], src_c)
            def scan_body(i, off):
                k = key_c[pl.ds(i * 16, 16)]
                m = (k >= lo8v) & (k < lo8v + P8)
                kl = k - lo8v
                pos = off + plsc.cumsum(m.astype(jnp.int32)) - 1
                pr = lax.shift_right_logical(pos, 7)
                pc = pos & (B - 1)
                plsc.store_scatter(cidx, [pr, pc], kl, mask=m)
                sv = src_c[pl.ds(i * 16, 16)]
                plsc.store_scatter(csrc, [pr, pc], sv, mask=m)
                plsc.addupdate_scatter(
                    cnt_v, [lax.shift_right_logical(kl, 7), kl & (C - 1)],
                    ones16f, mask=m)
                return off + plsc.all_reduce_population_count(m)
            return lax.fori_loop(0, ECH // 16, scan_body, off_c)
        off_v = lax.fori_loop(0, NCH, chunk_body, jnp.zeros((16,), jnp.int32))
        off = jnp.max(off_v)

        for j in range(B // 16):
            pos = off_v + (j * 16) + iota
            pr = lax.shift_right_logical(pos, 7)
            pc = pos & (B - 1)
            plsc.store_scatter(cidx, [pr, pc], dummyv)
            plsc.store_scatter(csrc, [pr, pc], zerov)

        pltpu.sync_copy(cnt_v, cnt_sh.at[ident], add=True)
        nb = lax.shift_right_logical(off + (B - 1), 7)
        def g_body(j, _):
            pltpu.async_copy(xa.at[csrc.at[j]], rows, sem).wait()
            pltpu.sync_copy(rows, acc.at[cidx.at[j]], add=True)
            return 0
        lax.fori_loop(0, nb, g_body, 0)
        plsc.subcore_barrier()

        obase = s * (P8 // NSUB)
        def o_body(j, _):
            pltpu.sync_copy(acc.at[pl.ds(obase + j * 56, 56)],
                            outh.at[rid, pl.ds(obase + j * 56, 56)])
            return 0
        lax.fori_loop(0, 7, o_body, 0)
        pltpu.sync_copy(acc.at[pl.ds(obase + 392, 32)],
                        outh.at[rid, pl.ds(obase + 392, 32)])
        @pl.when(s < CROWS // 8)
        def _():
            pltpu.sync_copy(cnt_sh.at[pl.ds(s * 8, 8)],
                            outch.at[rid, pl.ds(s * 8, 8)])
        plsc.subcore_barrier()


def _sc_scatter(xa, key, src, identa):
    mesh = plsc.VectorSubcoreMesh(core_axis_name="c", subcore_axis_name="s")
    return pl.kernel(
        _sc_body,
        mesh=mesh,
        compiler_params=pltpu.CompilerParams(needs_layout_passes=False),
        out_type=(
            jax.ShapeDtypeStruct((NRANGE, P8, C), jnp.float32),
            jax.ShapeDtypeStruct((NRANGE, CROWS, C), jnp.float32),
        ),
        scratch_types=[
            pltpu.VMEM((ECH,), jnp.int32),
            pltpu.VMEM((ECH,), jnp.int32),
            pltpu.VMEM((CAP // B, B), jnp.int32),
            pltpu.VMEM((CAP // B, B), jnp.int32),
            pltpu.VMEM((B, C), jnp.float32),
            pltpu.VMEM((ZROWS, C), jnp.float32),
            pltpu.VMEM((CROWS, C), jnp.float32),
            pltpu.VMEM((CROWS,), jnp.int32),
            pltpu.VMEM_SHARED((ACC_ROWS, C), jnp.float32),
            pltpu.VMEM_SHARED((CROWS, C), jnp.float32),
            pltpu.SemaphoreType.DMA,
        ],
    )(xa, key, src, identa)


def _tc_body(x_ref, s_ref, c_ref, w_ref, root_ref, bias_ref, o_ref):
    acc = jnp.dot(x_ref[...], root_ref[...],
                  preferred_element_type=jnp.float32) + bias_ref[...]
    cnt = jnp.maximum(c_ref[...], 1.0)
    for t in range(R):
        mean = s_ref[:, t, :] / cnt[:, t:t + 1]
        acc = acc + jnp.dot(mean, w_ref[t], preferred_element_type=jnp.float32)
    o_ref[...] = jnp.maximum(acc, 0.0)


def _tc_combine(x, s3, cnt2, weight, root, bias2):
    BN = 400
    return pl.pallas_call(
        _tc_body,
        grid=(N // BN,),
        in_specs=[
            pl.BlockSpec((BN, C), lambda i: (i, 0)),
            pl.BlockSpec((BN, R, C), lambda i: (i, 0, 0)),
            pl.BlockSpec((BN, R), lambda i: (i, 0)),
            pl.BlockSpec((R, C, C), lambda i: (0, 0, 0)),
            pl.BlockSpec((C, C), lambda i: (0, 0)),
            pl.BlockSpec((1, C), lambda i: (0, 0)),
        ],
        out_specs=pl.BlockSpec((BN, C), lambda i: (i, 0)),
        out_shape=jax.ShapeDtypeStruct((N, C), jnp.float32),
    )(x, s3, cnt2, weight, root, bias2)


@jax.jit
def kernel(node_features, edge_index, edge_type, weight, root, bias):
    x = node_features.astype(jnp.float32)
    src = edge_index[0].astype(jnp.int32)
    dst = edge_index[1].astype(jnp.int32)
    et = edge_type.astype(jnp.int32)
    key = dst * 8 + et
    identa = jnp.arange(CROWS, dtype=jnp.int32)
    sums, cnts = _sc_scatter(x, key, src, identa)
    s3 = sums.reshape(NRANGE * P, R, C)
    cnt2 = cnts.reshape(NRANGE, CROWS * C)[:, :P8].reshape(NRANGE * P, R)
    return _tc_combine(x, s3, cnt2, weight, root.astype(jnp.float32),
                       bias.reshape(1, C).astype(jnp.float32))


# speedup vs baseline: 9.8723x; 1.0771x over previous
import jax
import jax.numpy as jnp
from jax import lax
from jax.experimental import pallas as pl
from jax.experimental.pallas import tpu as pltpu
from jax.experimental.pallas import tpu_sc as plsc

N = 10000
E = 320000
C = 128
R = 8
P = 848
P8 = P * 8
NRANGE = 12
NPASS = 6
NSUB = 16
EPT = E // NSUB
ECH = 2000
NCH = EPT // ECH
CAP = 20480
B = 128
ACC_ROWS = 6912
DUMMY = P8
ZROWS = 32
P8DUP = 8192
CROWS = 56


def _sc_body(xa, keyh, srch, identh, outh, outch,
             key_c, src_c, cpack, idxb0, idxb1, srcb0, srcb1, rows0, rows1,
             zbuf, cnt_v, ident, acc, cnt_sh, sem0, sem1):
    c = lax.axis_index("c")
    s = lax.axis_index("s")

    pltpu.sync_copy(identh, ident)

    zero16f = jnp.zeros((16,), jnp.float32)
    for zr in range(ZROWS):
        for zc in range(C // 16):
            zbuf[zr, pl.ds(zc * 16, 16)] = zero16f

    iota = lax.iota(jnp.int32, 16)
    dummyv = jnp.full((16,), DUMMY, jnp.int32)
    zerov = jnp.zeros((16,), jnp.int32)
    ones16f = jnp.ones((16,), jnp.float32)

    for pass_ in range(NPASS):
        rid = pass_ * 2 + c
        lo8 = rid * P8

        zbase = s * (ACC_ROWS // NSUB)
        def z_body(j, _):
            pltpu.sync_copy(zbuf, acc.at[pl.ds(zbase + j * ZROWS, ZROWS)])
            return 0
        lax.fori_loop(0, 13, z_body, 0)
        pltpu.sync_copy(zbuf.at[pl.ds(0, 16)],
                        acc.at[pl.ds(zbase + 416, 16)])
        @pl.when(s < CROWS // 8)
        def _():
            pltpu.sync_copy(zbuf.at[pl.ds(0, 8)], cnt_sh.at[pl.ds(s * 8, 8)])
        def zc_body(j, _):
            for zc in range(C // 16):
                cnt_v[j, pl.ds(zc * 16, 16)] = zero16f
            return 0
        lax.fori_loop(0, CROWS, zc_body, 0)
        plsc.subcore_barrier()

        lo8v = jnp.full((16,), lo8, jnp.int32)
        def chunk_body(ch, off_c):
            ebase = s * EPT + ch * ECH
            pltpu.sync_copy(keyh.at[pl.ds(ebase, ECH)], key_c)
            pltpu.sync_copy(srch.at[pl.ds(ebase, ECH)], src_c)
            def scan_body(i, off):
                k = key_c[pl.ds(i * 16, 16)]
                m = (k >= lo8v) & (k < lo8v + P8)
                kl = k - lo8v
                pos = off + plsc.cumsum(m.astype(jnp.int32)) - 1
                pr = lax.shift_right_logical(pos, 7)
                pc = pos & (B - 1)
                sv = src_c[pl.ds(i * 16, 16)]
                plsc.store_scatter(cpack, [pr, pc],
                                   lax.shift_left(sv, 13) | kl, mask=m)
                plsc.addupdate_scatter(
                    cnt_v, [lax.shift_right_logical(kl, 7), kl & (C - 1)],
                    ones16f, mask=m)
                return off + plsc.all_reduce_population_count(m)
            return lax.fori_loop(0, ECH // 16, scan_body, off_c)
        off_v = lax.fori_loop(0, NCH, chunk_body, jnp.zeros((16,), jnp.int32))
        off = jnp.max(off_v)

        for j in range(B // 16):
            pos = off_v + (j * 16) + iota
            pr = lax.shift_right_logical(pos, 7)
            pc = pos & (B - 1)
            plsc.store_scatter(cpack, [pr, pc], dummyv)

        pltpu.sync_copy(cnt_v, cnt_sh.at[ident], add=True)
        nb = lax.shift_right_logical(off + (B - 1), 7)
        bufs = ((idxb0, srcb0, rows0, sem0), (idxb1, srcb1, rows1, sem1))

        def unpack(j, idxb, srcb):
            for cc in range(C // 16):
                v = cpack[j, pl.ds(cc * 16, 16)]
                srcb[pl.ds(cc * 16, 16)] = lax.shift_right_logical(v, 13)
                idxb[pl.ds(cc * 16, 16)] = v & (P8DUP - 1)

        @pl.when(nb > 0)
        def _():
            unpack(0, idxb0, srcb0)
            pltpu.async_copy(xa.at[srcb0], rows0, sem0)
        def g_outer(jo, _):
            for bsel in range(2):
                j = jo * 2 + bsel
                idxb, srcb, rows, sem = bufs[bsel]
                idxn, srcn, rowsn, semn = bufs[1 - bsel]
                @pl.when(j < nb)
                def _():
                    pltpu.make_async_copy(xa.at[srcb], rows, sem).wait()
                    @pl.when(j + 1 < nb)
                    def _():
                        unpack(j + 1, idxn, srcn)
                        pltpu.async_copy(xa.at[srcn], rowsn, semn)
                    pltpu.sync_copy(rows, acc.at[idxb], add=True)
            return 0
        lax.fori_loop(0, (nb + 1) >> 1, g_outer, 0)
        plsc.subcore_barrier()

        obase = s * (P8 // NSUB)
        def o_body(j, _):
            pltpu.sync_copy(acc.at[pl.ds(obase + j * 56, 56)],
                            outh.at[rid, pl.ds(obase + j * 56, 56)])
            return 0
        lax.fori_loop(0, 7, o_body, 0)
        pltpu.sync_copy(acc.at[pl.ds(obase + 392, 32)],
                        outh.at[rid, pl.ds(obase + 392, 32)])
        @pl.when(s < CROWS // 8)
        def _():
            pltpu.sync_copy(cnt_sh.at[pl.ds(s * 8, 8)],
                            outch.at[rid, pl.ds(s * 8, 8)])
        plsc.subcore_barrier()


def _sc_scatter(xa, key, src, identa):
    mesh = plsc.VectorSubcoreMesh(core_axis_name="c", subcore_axis_name="s")
    return pl.kernel(
        _sc_body,
        mesh=mesh,
        compiler_params=pltpu.CompilerParams(needs_layout_passes=False),
        out_type=(
            jax.ShapeDtypeStruct((NRANGE, P8, C), jnp.float32),
            jax.ShapeDtypeStruct((NRANGE, CROWS, C), jnp.float32),
        ),
        scratch_types=[
            pltpu.VMEM((ECH,), jnp.int32),
            pltpu.VMEM((ECH,), jnp.int32),
            pltpu.VMEM((CAP // B, B), jnp.int32),
            pltpu.VMEM((B,), jnp.int32),
            pltpu.VMEM((B,), jnp.int32),
            pltpu.VMEM((B,), jnp.int32),
            pltpu.VMEM((B,), jnp.int32),
            pltpu.VMEM((B, C), jnp.float32),
            pltpu.VMEM((B, C), jnp.float32),
            pltpu.VMEM((ZROWS, C), jnp.float32),
            pltpu.VMEM((CROWS, C), jnp.float32),
            pltpu.VMEM((CROWS,), jnp.int32),
            pltpu.VMEM_SHARED((ACC_ROWS, C), jnp.float32),
            pltpu.VMEM_SHARED((CROWS, C), jnp.float32),
            pltpu.SemaphoreType.DMA,
            pltpu.SemaphoreType.DMA,
        ],
    )(xa, key, src, identa)


def _tc_body(x_ref, s_ref, c_ref, w_ref, root_ref, bias_ref, o_ref):
    acc = jnp.dot(x_ref[...], root_ref[...],
                  preferred_element_type=jnp.float32) + bias_ref[...]
    cnt = jnp.maximum(c_ref[...], 1.0)
    for t in range(R):
        mean = s_ref[:, t, :] / cnt[:, t:t + 1]
        acc = acc + jnp.dot(mean, w_ref[t], preferred_element_type=jnp.float32)
    o_ref[...] = jnp.maximum(acc, 0.0)


def _tc_combine(x, s3, cnt2, weight, root, bias2):
    BN = 400
    return pl.pallas_call(
        _tc_body,
        grid=(N // BN,),
        in_specs=[
            pl.BlockSpec((BN, C), lambda i: (i, 0)),
            pl.BlockSpec((BN, R, C), lambda i: (i, 0, 0)),
            pl.BlockSpec((BN, R), lambda i: (i, 0)),
            pl.BlockSpec((R, C, C), lambda i: (0, 0, 0)),
            pl.BlockSpec((C, C), lambda i: (0, 0)),
            pl.BlockSpec((1, C), lambda i: (0, 0)),
        ],
        out_specs=pl.BlockSpec((BN, C), lambda i: (i, 0)),
        out_shape=jax.ShapeDtypeStruct((N, C), jnp.float32),
    )(x, s3, cnt2, weight, root, bias2)


@jax.jit
def kernel(node_features, edge_index, edge_type, weight, root, bias):
    x = node_features.astype(jnp.float32)
    src = edge_index[0].astype(jnp.int32)
    dst = edge_index[1].astype(jnp.int32)
    et = edge_type.astype(jnp.int32)
    key = dst * 8 + et
    identa = jnp.arange(CROWS, dtype=jnp.int32)
    sums, cnts = _sc_scatter(x, key, src, identa)
    s3 = sums.reshape(NRANGE * P, R, C)
    cnt2 = cnts.reshape(NRANGE, CROWS * C)[:, :P8].reshape(NRANGE * P, R)
    return _tc_combine(x, s3, cnt2, weight, root.astype(jnp.float32),
                       bias.reshape(1, C).astype(jnp.float32))


# speedup vs baseline: 10.9392x; 1.1081x over previous
import jax
import jax.numpy as jnp
from jax import lax
from jax.experimental import pallas as pl
from jax.experimental.pallas import tpu as pltpu
from jax.experimental.pallas import tpu_sc as plsc

N = 10000
E = 320000
C = 128
R = 8
P = 848
P8 = P * 8
NRANGE = 12
NPASS = 6
NSUB = 16
EPT = E // NSUB
ECH = 2000
NCH = EPT // ECH
CAP = 20480
B = 128
ACC_ROWS = 6912
DUMMY = P8
ZROWS = 32
P8DUP = 8192
CROWS = 56


def _sc_body(xa, keyh, srch, identh, outh, outch,
             key_c0, key_c1, src_c0, src_c1, cpack, idxb0, idxb1,
             srcb0, srcb1, rows0, rows1, zbuf, cnt_v, ident, acc, cnt_sh,
             sem0, sem1, szero, schunk, souts):
    c = lax.axis_index("c")
    s = lax.axis_index("s")

    pltpu.sync_copy(identh, ident)

    zero16f = jnp.zeros((16,), jnp.float32)
    for zr in range(ZROWS):
        for zc in range(C // 16):
            zbuf[zr, pl.ds(zc * 16, 16)] = zero16f

    iota = lax.iota(jnp.int32, 16)
    dummyv = jnp.full((16,), DUMMY, jnp.int32)
    zerov = jnp.zeros((16,), jnp.int32)
    ones16f = jnp.ones((16,), jnp.float32)

    for pass_ in range(NPASS):
        rid = pass_ * 2 + c
        lo8 = rid * P8

        zbase = s * (ACC_ROWS // NSUB)
        for j in range(13):
            pltpu.async_copy(zbuf, acc.at[pl.ds(zbase + j * ZROWS, ZROWS)],
                             szero)
        pltpu.async_copy(zbuf.at[pl.ds(0, 16)],
                         acc.at[pl.ds(zbase + 416, 16)], szero)
        @pl.when(s < CROWS // 8)
        def _():
            pltpu.async_copy(zbuf.at[pl.ds(0, 8)], cnt_sh.at[pl.ds(s * 8, 8)],
                             szero)
        def zc_body(j, _):
            for zc in range(C // 16):
                cnt_v[j, pl.ds(zc * 16, 16)] = zero16f
            return 0
        lax.fori_loop(0, CROWS, zc_body, 0)

        lo8v = jnp.full((16,), lo8, jnp.int32)
        ebufs = ((key_c0, src_c0), (key_c1, src_c1))

        def fire_chunk(ch, bsel):
            ebase = s * EPT + ch * ECH
            pltpu.async_copy(keyh.at[pl.ds(ebase, ECH)], ebufs[bsel][0], schunk)
            pltpu.async_copy(srch.at[pl.ds(ebase, ECH)], ebufs[bsel][1], schunk)

        def wait_chunk(ch, bsel):
            ebase = s * EPT + ch * ECH
            pltpu.make_async_copy(keyh.at[pl.ds(ebase, ECH)], ebufs[bsel][0],
                                  schunk).wait()
            pltpu.make_async_copy(srch.at[pl.ds(ebase, ECH)], ebufs[bsel][1],
                                  schunk).wait()

        def make_scan_body(key_c, src_c):
            def scan_body(i, off):
                for u in range(5):
                    k = key_c[pl.ds(i * 80 + u * 16, 16)]
                    m = (k >= lo8v) & (k < lo8v + P8)
                    kl = k - lo8v
                    pos = off + plsc.cumsum(m.astype(jnp.int32)) - 1
                    pr = lax.shift_right_logical(pos, 7)
                    pc = pos & (B - 1)
                    sv = src_c[pl.ds(i * 80 + u * 16, 16)]
                    plsc.store_scatter(cpack, [pr, pc],
                                       lax.shift_left(sv, 13) | kl, mask=m)
                    plsc.addupdate_scatter(
                        cnt_v, [lax.shift_right_logical(kl, 7), kl & (C - 1)],
                        ones16f, mask=m)
                    off = off + plsc.all_reduce_population_count(m)
                return off
            return scan_body

        fire_chunk(0, 0)
        def chunks_body(co, off_c):
            for bsel in range(2):
                ch = co * 2 + bsel
                wait_chunk(ch, bsel)
                @pl.when(ch + 1 < NCH)
                def _():
                    fire_chunk(ch + 1, 1 - bsel)
                off_c = lax.fori_loop(
                    0, ECH // 80, make_scan_body(*ebufs[bsel]), off_c)
            return off_c
        off_v = lax.fori_loop(0, NCH // 2, chunks_body,
                              jnp.zeros((16,), jnp.int32))
        off = jnp.max(off_v)

        for j in range(13):
            pltpu.make_async_copy(
                zbuf, acc.at[pl.ds(zbase + j * ZROWS, ZROWS)], szero).wait()
        pltpu.make_async_copy(zbuf.at[pl.ds(0, 16)],
                              acc.at[pl.ds(zbase + 416, 16)], szero).wait()
        @pl.when(s < CROWS // 8)
        def _():
            pltpu.make_async_copy(zbuf.at[pl.ds(0, 8)],
                                  cnt_sh.at[pl.ds(s * 8, 8)], szero).wait()
        plsc.subcore_barrier()

        for j in range(B // 16):
            pos = off_v + (j * 16) + iota
            pr = lax.shift_right_logical(pos, 7)
            pc = pos & (B - 1)
            plsc.store_scatter(cpack, [pr, pc], dummyv)

        pltpu.sync_copy(cnt_v, cnt_sh.at[ident], add=True)
        nb = lax.shift_right_logical(off + (B - 1), 7)
        bufs = ((idxb0, srcb0, rows0, sem0), (idxb1, srcb1, rows1, sem1))

        def unpack(j, idxb, srcb):
            for cc in range(C // 16):
                v = cpack[j, pl.ds(cc * 16, 16)]
                srcb[pl.ds(cc * 16, 16)] = lax.shift_right_logical(v, 13)
                idxb[pl.ds(cc * 16, 16)] = v & (P8DUP - 1)

        @pl.when(nb > 0)
        def _():
            unpack(0, idxb0, srcb0)
            pltpu.async_copy(xa.at[srcb0], rows0, sem0)
        def g_outer(jo, _):
            for bsel in range(2):
                j = jo * 2 + bsel
                idxb, srcb, rows, sem = bufs[bsel]
                idxn, srcn, rowsn, semn = bufs[1 - bsel]
                @pl.when(j < nb)
                def _():
                    pltpu.make_async_copy(xa.at[srcb], rows, sem).wait()
                    @pl.when(j + 1 < nb)
                    def _():
                        unpack(j + 1, idxn, srcn)
                        pltpu.async_copy(xa.at[srcn], rowsn, semn)
                    pltpu.sync_copy(rows, acc.at[idxb], add=True)
            return 0
        lax.fori_loop(0, (nb + 1) >> 1, g_outer, 0)
        plsc.subcore_barrier()

        obase = s * (P8 // NSUB)
        for j in range(7):
            pltpu.async_copy(acc.at[pl.ds(obase + j * 56, 56)],
                             outh.at[rid, pl.ds(obase + j * 56, 56)], souts)
        pltpu.async_copy(acc.at[pl.ds(obase + 392, 32)],
                         outh.at[rid, pl.ds(obase + 392, 32)], souts)
        @pl.when(s < CROWS // 8)
        def _():
            pltpu.async_copy(cnt_sh.at[pl.ds(s * 8, 8)],
                             outch.at[rid, pl.ds(s * 8, 8)], souts)
        for j in range(7):
            pltpu.make_async_copy(
                acc.at[pl.ds(obase + j * 56, 56)],
                outh.at[rid, pl.ds(obase + j * 56, 56)], souts).wait()
        pltpu.make_async_copy(acc.at[pl.ds(obase + 392, 32)],
                              outh.at[rid, pl.ds(obase + 392, 32)],
                              souts).wait()
        @pl.when(s < CROWS // 8)
        def _():
            pltpu.make_async_copy(cnt_sh.at[pl.ds(s * 8, 8)],
                                  outch.at[rid, pl.ds(s * 8, 8)],
                                  souts).wait()
        plsc.subcore_barrier()


def _sc_scatter(xa, key, src, identa):
    mesh = plsc.VectorSubcoreMesh(core_axis_name="c", subcore_axis_name="s")
    return pl.kernel(
        _sc_body,
        mesh=mesh,
        compiler_params=pltpu.CompilerParams(needs_layout_passes=False),
        out_type=(
            jax.ShapeDtypeStruct((NRANGE, P8, C), jnp.float32),
            jax.ShapeDtypeStruct((NRANGE, CROWS, C), jnp.float32),
        ),
        scratch_types=[
            pltpu.VMEM((ECH,), jnp.int32),
            pltpu.VMEM((ECH,), jnp.int32),
            pltpu.VMEM((ECH,), jnp.int32),
            pltpu.VMEM((ECH,), jnp.int32),
            pltpu.VMEM((CAP // B, B), jnp.int32),
            pltpu.VMEM((B,), jnp.int32),
            pltpu.VMEM((B,), jnp.int32),
            pltpu.VMEM((B,), jnp.int32),
            pltpu.VMEM((B,), jnp.int32),
            pltpu.VMEM((B, C), jnp.float32),
            pltpu.VMEM((B, C), jnp.float32),
            pltpu.VMEM((ZROWS, C), jnp.float32),
            pltpu.VMEM((CROWS, C), jnp.float32),
            pltpu.VMEM((CROWS,), jnp.int32),
            pltpu.VMEM_SHARED((ACC_ROWS, C), jnp.float32),
            pltpu.VMEM_SHARED((CROWS, C), jnp.float32),
            pltpu.SemaphoreType.DMA,
            pltpu.SemaphoreType.DMA,
            pltpu.SemaphoreType.DMA,
            pltpu.SemaphoreType.DMA,
            pltpu.SemaphoreType.DMA,
        ],
    )(xa, key, src, identa)


def _tc_body(x_ref, s_ref, c_ref, w_ref, root_ref, bias_ref, o_ref):
    acc = jnp.dot(x_ref[...], root_ref[...],
                  preferred_element_type=jnp.float32) + bias_ref[...]
    cnt = jnp.maximum(c_ref[...], 1.0)
    for t in range(R):
        mean = s_ref[:, t, :] / cnt[:, t:t + 1]
        acc = acc + jnp.dot(mean, w_ref[t], preferred_element_type=jnp.float32)
    o_ref[...] = jnp.maximum(acc, 0.0)


def _tc_combine(x, s3, cnt2, weight, root, bias2):
    BN = 400
    return pl.pallas_call(
        _tc_body,
        grid=(N // BN,),
        in_specs=[
            pl.BlockSpec((BN, C), lambda i: (i, 0)),
            pl.BlockSpec((BN, R, C), lambda i: (i, 0, 0)),
            pl.BlockSpec((BN, R), lambda i: (i, 0)),
            pl.BlockSpec((R, C, C), lambda i: (0, 0, 0)),
            pl.BlockSpec((C, C), lambda i: (0, 0)),
            pl.BlockSpec((1, C), lambda i: (0, 0)),
        ],
        out_specs=pl.BlockSpec((BN, C), lambda i: (i, 0)),
        out_shape=jax.ShapeDtypeStruct((N, C), jnp.float32),
    )(x, s3, cnt2, weight, root, bias2)


@jax.jit
def kernel(node_features, edge_index, edge_type, weight, root, bias):
    x = node_features.astype(jnp.float32)
    src = edge_index[0].astype(jnp.int32)
    dst = edge_index[1].astype(jnp.int32)
    et = edge_type.astype(jnp.int32)
    key = dst * 8 + et
    identa = jnp.arange(CROWS, dtype=jnp.int32)
    sums, cnts = _sc_scatter(x, key, src, identa)
    s3 = sums.reshape(NRANGE * P, R, C)
    cnt2 = cnts.reshape(NRANGE, CROWS * C)[:, :P8].reshape(NRANGE * P, R)
    return _tc_combine(x, s3, cnt2, weight, root.astype(jnp.float32),
                       bias.reshape(1, C).astype(jnp.float32))


# speedup vs baseline: 10.9714x; 1.0029x over previous
import jax
import jax.numpy as jnp
from jax import lax
from jax.experimental import pallas as pl
from jax.experimental.pallas import tpu as pltpu
from jax.experimental.pallas import tpu_sc as plsc

N = 10000
E = 320000
C = 128
R = 8
P = 848
P8 = P * 8
NRANGE = 12
NPASS = 6
NSUB = 16
EPT = E // NSUB
ECH = 2000
NCH = EPT // ECH
CAP = 20480
B = 128
ACC_ROWS = 6912
DUMMY = P8
ZROWS = 32
P8DUP = 8192
CROWS = 56


def _sc_body(xa, keyh, srch, identh, outh, outch,
             key_c0, key_c1, src_c0, src_c1, cpack, idxb0, idxb1,
             srcb0, srcb1, rows0, rows1, zbuf, cnt_v, ident, acc, cnt_sh,
             sem0, sem1, scat0, scat1, szero, schunk, souts):
    c = lax.axis_index("c")
    s = lax.axis_index("s")

    pltpu.sync_copy(identh, ident)

    zero16f = jnp.zeros((16,), jnp.float32)
    for zr in range(ZROWS):
        for zc in range(C // 16):
            zbuf[zr, pl.ds(zc * 16, 16)] = zero16f

    iota = lax.iota(jnp.int32, 16)
    dummyv = jnp.full((16,), DUMMY, jnp.int32)
    zerov = jnp.zeros((16,), jnp.int32)
    ones16f = jnp.ones((16,), jnp.float32)

    for pass_ in range(NPASS):
        rid = pass_ * 2 + c
        lo8 = rid * P8

        zbase = s * (ACC_ROWS // NSUB)
        for j in range(13):
            pltpu.async_copy(zbuf, acc.at[pl.ds(zbase + j * ZROWS, ZROWS)],
                             szero)
        pltpu.async_copy(zbuf.at[pl.ds(0, 16)],
                         acc.at[pl.ds(zbase + 416, 16)], szero)
        @pl.when(s < CROWS // 8)
        def _():
            pltpu.async_copy(zbuf.at[pl.ds(0, 8)], cnt_sh.at[pl.ds(s * 8, 8)],
                             szero)
        def zc_body(j, _):
            for zc in range(C // 16):
                cnt_v[j, pl.ds(zc * 16, 16)] = zero16f
            return 0
        lax.fori_loop(0, CROWS, zc_body, 0)

        lo8v = jnp.full((16,), lo8, jnp.int32)
        ebufs = ((key_c0, src_c0), (key_c1, src_c1))

        def fire_chunk(ch, bsel):
            ebase = s * EPT + ch * ECH
            pltpu.async_copy(keyh.at[pl.ds(ebase, ECH)], ebufs[bsel][0], schunk)
            pltpu.async_copy(srch.at[pl.ds(ebase, ECH)], ebufs[bsel][1], schunk)

        def wait_chunk(ch, bsel):
            ebase = s * EPT + ch * ECH
            pltpu.make_async_copy(keyh.at[pl.ds(ebase, ECH)], ebufs[bsel][0],
                                  schunk).wait()
            pltpu.make_async_copy(srch.at[pl.ds(ebase, ECH)], ebufs[bsel][1],
                                  schunk).wait()

        def make_scan_body(key_c, src_c):
            def scan_body(i, off):
                for u in range(5):
                    k = key_c[pl.ds(i * 80 + u * 16, 16)]
                    m = (k >= lo8v) & (k < lo8v + P8)
                    kl = k - lo8v
                    pos = off + plsc.cumsum(m.astype(jnp.int32)) - 1
                    pr = lax.shift_right_logical(pos, 7)
                    pc = pos & (B - 1)
                    sv = src_c[pl.ds(i * 80 + u * 16, 16)]
                    plsc.store_scatter(cpack, [pr, pc],
                                       lax.shift_left(sv, 13) | kl, mask=m)
                    plsc.addupdate_scatter(
                        cnt_v, [lax.shift_right_logical(kl, 7), kl & (C - 1)],
                        ones16f, mask=m)
                    off = off + plsc.all_reduce_population_count(m)
                return off
            return scan_body

        fire_chunk(0, 0)
        def chunks_body(co, off_c):
            for bsel in range(2):
                ch = co * 2 + bsel
                wait_chunk(ch, bsel)
                @pl.when(ch + 1 < NCH)
                def _():
                    fire_chunk(ch + 1, 1 - bsel)
                off_c = lax.fori_loop(
                    0, ECH // 80, make_scan_body(*ebufs[bsel]), off_c)
            return off_c
        off_v = lax.fori_loop(0, NCH // 2, chunks_body,
                              jnp.zeros((16,), jnp.int32))
        off = jnp.max(off_v)

        for j in range(13):
            pltpu.make_async_copy(
                zbuf, acc.at[pl.ds(zbase + j * ZROWS, ZROWS)], szero).wait()
        pltpu.make_async_copy(zbuf.at[pl.ds(0, 16)],
                              acc.at[pl.ds(zbase + 416, 16)], szero).wait()
        @pl.when(s < CROWS // 8)
        def _():
            pltpu.make_async_copy(zbuf.at[pl.ds(0, 8)],
                                  cnt_sh.at[pl.ds(s * 8, 8)], szero).wait()
        plsc.subcore_barrier()

        for j in range(B // 16):
            pos = off_v + (j * 16) + iota
            pr = lax.shift_right_logical(pos, 7)
            pc = pos & (B - 1)
            plsc.store_scatter(cpack, [pr, pc], dummyv)

        pltpu.async_copy(cnt_v, cnt_sh.at[ident], souts, add=True)
        nb = lax.shift_right_logical(off + (B - 1), 7)
        bufs = ((idxb0, srcb0, rows0, sem0, scat0),
                (idxb1, srcb1, rows1, sem1, scat1))

        def unpack(j, idxb, srcb):
            for cc in range(C // 16):
                v = cpack[j, pl.ds(cc * 16, 16)]
                srcb[pl.ds(cc * 16, 16)] = lax.shift_right_logical(v, 13)
                idxb[pl.ds(cc * 16, 16)] = v & (P8DUP - 1)

        @pl.when(nb > 0)
        def _():
            unpack(0, idxb0, srcb0)
            pltpu.async_copy(xa.at[srcb0], rows0, sem0)
        def g_outer(jo, _):
            for bsel in range(2):
                j = jo * 2 + bsel
                idxb, srcb, rows, sem, scat = bufs[bsel]
                idxn, srcn, rowsn, semn, scatn = bufs[1 - bsel]
                @pl.when(j < nb)
                def _():
                    pltpu.make_async_copy(xa.at[srcb], rows, sem).wait()
                    pltpu.async_copy(rows, acc.at[idxb], scat, add=True)
                    @pl.when(j >= 1)
                    def _():
                        pltpu.make_async_copy(rowsn, acc.at[idxn],
                                              scatn).wait()
                    @pl.when(j + 1 < nb)
                    def _():
                        unpack(j + 1, idxn, srcn)
                        pltpu.async_copy(xa.at[srcn], rowsn, semn)
            return 0
        lax.fori_loop(0, (nb + 1) >> 1, g_outer, 0)
        @pl.when((nb > 0) & (((nb - 1) & 1) == 0))
        def _():
            pltpu.make_async_copy(rows0, acc.at[idxb0], scat0).wait()
        @pl.when((nb > 0) & (((nb - 1) & 1) == 1))
        def _():
            pltpu.make_async_copy(rows1, acc.at[idxb1], scat1).wait()
        pltpu.make_async_copy(cnt_v, cnt_sh.at[ident], souts).wait()
        plsc.subcore_barrier()

        obase = s * (P8 // NSUB)
        for j in range(7):
            pltpu.async_copy(acc.at[pl.ds(obase + j * 56, 56)],
                             outh.at[rid, pl.ds(obase + j * 56, 56)], souts)
        pltpu.async_copy(acc.at[pl.ds(obase + 392, 32)],
                         outh.at[rid, pl.ds(obase + 392, 32)], souts)
        @pl.when(s < CROWS // 8)
        def _():
            pltpu.async_copy(cnt_sh.at[pl.ds(s * 8, 8)],
                             outch.at[rid, pl.ds(s * 8, 8)], souts)
        for j in range(7):
            pltpu.make_async_copy(
                acc.at[pl.ds(obase + j * 56, 56)],
                outh.at[rid, pl.ds(obase + j * 56, 56)], souts).wait()
        pltpu.make_async_copy(acc.at[pl.ds(obase + 392, 32)],
                              outh.at[rid, pl.ds(obase + 392, 32)],
                              souts).wait()
        @pl.when(s < CROWS // 8)
        def _():
            pltpu.make_async_copy(cnt_sh.at[pl.ds(s * 8, 8)],
                                  outch.at[rid, pl.ds(s * 8, 8)],
                                  souts).wait()
        plsc.subcore_barrier()


def _sc_scatter(xa, key, src, identa):
    mesh = plsc.VectorSubcoreMesh(core_axis_name="c", subcore_axis_name="s")
    return pl.kernel(
        _sc_body,
        mesh=mesh,
        compiler_params=pltpu.CompilerParams(needs_layout_passes=False),
        out_type=(
            jax.ShapeDtypeStruct((NRANGE, P8, C), jnp.float32),
            jax.ShapeDtypeStruct((NRANGE, CROWS, C), jnp.float32),
        ),
        scratch_types=[
            pltpu.VMEM((ECH,), jnp.int32),
            pltpu.VMEM((ECH,), jnp.int32),
            pltpu.VMEM((ECH,), jnp.int32),
            pltpu.VMEM((ECH,), jnp.int32),
            pltpu.VMEM((CAP // B, B), jnp.int32),
            pltpu.VMEM((B,), jnp.int32),
            pltpu.VMEM((B,), jnp.int32),
            pltpu.VMEM((B,), jnp.int32),
            pltpu.VMEM((B,), jnp.int32),
            pltpu.VMEM((B, C), jnp.float32),
            pltpu.VMEM((B, C), jnp.float32),
            pltpu.VMEM((ZROWS, C), jnp.float32),
            pltpu.VMEM((CROWS, C), jnp.float32),
            pltpu.VMEM((CROWS,), jnp.int32),
            pltpu.VMEM_SHARED((ACC_ROWS, C), jnp.float32),
            pltpu.VMEM_SHARED((CROWS, C), jnp.float32),
            pltpu.SemaphoreType.DMA,
            pltpu.SemaphoreType.DMA,
            pltpu.SemaphoreType.DMA,
            pltpu.SemaphoreType.DMA,
            pltpu.SemaphoreType.DMA,
            pltpu.SemaphoreType.DMA,
            pltpu.SemaphoreType.DMA,
        ],
    )(xa, key, src, identa)


def _tc_body(x_ref, s_ref, c_ref, w_ref, root_ref, bias_ref, o_ref):
    acc = jnp.dot(x_ref[...], root_ref[...],
                  preferred_element_type=jnp.float32) + bias_ref[...]
    cnt = jnp.maximum(c_ref[...], 1.0)
    for t in range(R):
        mean = s_ref[:, t, :] / cnt[:, t:t + 1]
        acc = acc + jnp.dot(mean, w_ref[t], preferred_element_type=jnp.float32)
    o_ref[...] = jnp.maximum(acc, 0.0)


def _tc_combine(x, s3, cnt2, weight, root, bias2):
    BN = 400
    return pl.pallas_call(
        _tc_body,
        grid=(N // BN,),
        in_specs=[
            pl.BlockSpec((BN, C), lambda i: (i, 0)),
            pl.BlockSpec((BN, R, C), lambda i: (i, 0, 0)),
            pl.BlockSpec((BN, R), lambda i: (i, 0)),
            pl.BlockSpec((R, C, C), lambda i: (0, 0, 0)),
            pl.BlockSpec((C, C), lambda i: (0, 0)),
            pl.BlockSpec((1, C), lambda i: (0, 0)),
        ],
        out_specs=pl.BlockSpec((BN, C), lambda i: (i, 0)),
        out_shape=jax.ShapeDtypeStruct((N, C), jnp.float32),
    )(x, s3, cnt2, weight, root, bias2)


@jax.jit
def kernel(node_features, edge_index, edge_type, weight, root, bias):
    x = node_features.astype(jnp.float32)
    src = edge_index[0].astype(jnp.int32)
    dst = edge_index[1].astype(jnp.int32)
    et = edge_type.astype(jnp.int32)
    key = dst * 8 + et
    identa = jnp.arange(CROWS, dtype=jnp.int32)
    sums, cnts = _sc_scatter(x, key, src, identa)
    s3 = sums.reshape(NRANGE * P, R, C)
    cnt2 = cnts.reshape(NRANGE, CROWS * C)[:, :P8].reshape(NRANGE * P, R)
    return _tc_combine(x, s3, cnt2, weight, root.astype(jnp.float32),
                       bias.reshape(1, C).astype(jnp.float32))


# speedup vs baseline: 11.8505x; 1.0801x over previous
import jax
import jax.numpy as jnp
from jax import lax
from jax.experimental import pallas as pl
from jax.experimental.pallas import tpu as pltpu
from jax.experimental.pallas import tpu_sc as plsc

N = 10000
E = 320000
C = 128
R = 8
P = 848
P8 = P * 8
NRANGE = 12
NPASS = 6
NSUB = 16
EPT = E // NSUB
ECH = 2000
NCH = EPT // ECH
CAP = 20480
B = 128
ACC_ROWS = 6912
DUMMY = P8
ZROWS = 32
P8DUP = 8192
CROWS = 56


def _sc_body(xa, keyh, srch, identh, outh, outch,
             key_c0, key_c1, src_c0, src_c1, cpack, idxb0, idxb1,
             srcb0, srcb1, rows0, rows1, zbuf, cnt_v, ident, acc, cnt_sh,
             sem0, sem1, scat0, scat1, szero, schunk, souts):
    c = lax.axis_index("c")
    s = lax.axis_index("s")

    pltpu.sync_copy(identh, ident)

    zero16f = jnp.zeros((16,), jnp.float32)
    for zr in range(ZROWS):
        for zc in range(C // 16):
            zbuf[zr, pl.ds(zc * 16, 16)] = zero16f

    iota = lax.iota(jnp.int32, 16)
    dummyv = jnp.full((16,), DUMMY, jnp.int32)
    zerov = jnp.zeros((16,), jnp.int32)
    ones16f = jnp.ones((16,), jnp.float32)

    for pass_ in range(NPASS):
        rid = pass_ * 2 + c
        lo8 = rid * P8

        zbase = s * (ACC_ROWS // NSUB)
        for j in range(13):
            pltpu.async_copy(zbuf, acc.at[pl.ds(zbase + j * ZROWS, ZROWS)],
                             szero)
        pltpu.async_copy(zbuf.at[pl.ds(0, 16)],
                         acc.at[pl.ds(zbase + 416, 16)], szero)
        @pl.when(s < CROWS // 8)
        def _():
            pltpu.async_copy(zbuf.at[pl.ds(0, 8)], cnt_sh.at[pl.ds(s * 8, 8)],
                             szero)

        lo8v = jnp.full((16,), lo8, jnp.int32)
        ebufs = ((key_c0, src_c0), (key_c1, src_c1))

        def fire_chunk(ch, bsel):
            ebase = s * EPT + ch * ECH
            pltpu.async_copy(keyh.at[pl.ds(ebase, ECH)], ebufs[bsel][0], schunk)
            pltpu.async_copy(srch.at[pl.ds(ebase, ECH)], ebufs[bsel][1], schunk)

        def wait_chunk(ch, bsel):
            ebase = s * EPT + ch * ECH
            pltpu.make_async_copy(keyh.at[pl.ds(ebase, ECH)], ebufs[bsel][0],
                                  schunk).wait()
            pltpu.make_async_copy(srch.at[pl.ds(ebase, ECH)], ebufs[bsel][1],
                                  schunk).wait()

        fire_chunk(0, 0)

        def zc_body(j, _):
            for zc in range(C // 16):
                cnt_v[j, pl.ds(zc * 16, 16)] = zero16f
            return 0
        lax.fori_loop(0, CROWS, zc_body, 0)

        for j in range(13):
            pltpu.make_async_copy(
                zbuf, acc.at[pl.ds(zbase + j * ZROWS, ZROWS)], szero).wait()
        pltpu.make_async_copy(zbuf.at[pl.ds(0, 16)],
                              acc.at[pl.ds(zbase + 416, 16)], szero).wait()
        @pl.when(s < CROWS // 8)
        def _():
            pltpu.make_async_copy(zbuf.at[pl.ds(0, 8)],
                                  cnt_sh.at[pl.ds(s * 8, 8)], szero).wait()
        plsc.subcore_barrier()

        def unpack(j, idxb, srcb):
            for cc in range(C // 16):
                v = cpack[j, pl.ds(cc * 16, 16)]
                srcb[pl.ds(cc * 16, 16)] = lax.shift_right_logical(v, 13)
                idxb[pl.ds(cc * 16, 16)] = v & (P8DUP - 1)

        def step_impl(j, idxb, srcb, rows, sem, scat,
                      idxo, srco, rowso, semo, scato):
            @pl.when(j >= 2)
            def _():
                pltpu.make_async_copy(rows, acc.at[idxb], scat).wait()
            unpack(j, idxb, srcb)
            pltpu.async_copy(xa.at[srcb], rows, sem)
            @pl.when(j >= 1)
            def _():
                pltpu.make_async_copy(xa.at[srco], rowso, semo).wait()
                pltpu.async_copy(rowso, acc.at[idxo], scato, add=True)

        def issue_step(j, carry):
            @pl.when((j & 1) == 0)
            def _():
                step_impl(j, idxb0, srcb0, rows0, sem0, scat0,
                          idxb1, srcb1, rows1, sem1, scat1)
            @pl.when((j & 1) == 1)
            def _():
                step_impl(j, idxb1, srcb1, rows1, sem1, scat1,
                          idxb0, srcb0, rows0, sem0, scat0)
            return carry

        def make_scan_body(key_c, src_c):
            def scan_body(i, off):
                for u in range(5):
                    k = key_c[pl.ds(i * 80 + u * 16, 16)]
                    m = (k >= lo8v) & (k < lo8v + P8)
                    kl = k - lo8v
                    pos = off + plsc.cumsum(m.astype(jnp.int32)) - 1
                    pr = lax.shift_right_logical(pos, 7)
                    pc = pos & (B - 1)
                    sv = src_c[pl.ds(i * 80 + u * 16, 16)]
                    plsc.store_scatter(cpack, [pr, pc],
                                       lax.shift_left(sv, 13) | kl, mask=m)
                    plsc.addupdate_scatter(
                        cnt_v, [lax.shift_right_logical(kl, 7), kl & (C - 1)],
                        ones16f, mask=m)
                    off = off + plsc.all_reduce_population_count(m)
                return off
            return scan_body

        def chunks_body(co, carry):
            off_c, done = carry
            for bsel in range(2):
                ch = co * 2 + bsel
                wait_chunk(ch, bsel)
                @pl.when(ch + 1 < NCH)
                def _():
                    fire_chunk(ch + 1, 1 - bsel)
                off_c = lax.fori_loop(
                    0, ECH // 80, make_scan_body(*ebufs[bsel]), off_c)
                ready = lax.shift_right_logical(jnp.max(off_c), 7)
                lax.fori_loop(done, ready, issue_step, 0)
                done = ready
            return (off_c, done)
        off_v, done = lax.fori_loop(
            0, NCH // 2, chunks_body,
            (jnp.zeros((16,), jnp.int32), jnp.int32(0)))
        off = jnp.max(off_v)

        pltpu.async_copy(cnt_v, cnt_sh.at[ident], souts, add=True)

        for j in range(B // 16):
            pos = off_v + (j * 16) + iota
            pr = lax.shift_right_logical(pos, 7)
            pc = pos & (B - 1)
            plsc.store_scatter(cpack, [pr, pc], dummyv)
        nb = lax.shift_right_logical(off + (B - 1), 7)
        lax.fori_loop(done, nb, issue_step, 0)
        @pl.when((nb > 0) & (((nb - 1) & 1) == 0))
        def _():
            pltpu.make_async_copy(xa.at[srcb0], rows0, sem0).wait()
            pltpu.async_copy(rows0, acc.at[idxb0], scat0, add=True)
        @pl.when((nb > 0) & (((nb - 1) & 1) == 1))
        def _():
            pltpu.make_async_copy(xa.at[srcb1], rows1, sem1).wait()
            pltpu.async_copy(rows1, acc.at[idxb1], scat1, add=True)
        @pl.when(nb >= 2)
        def _():
            @pl.when(((nb - 2) & 1) == 0)
            def _():
                pltpu.make_async_copy(rows0, acc.at[idxb0], scat0).wait()
            @pl.when(((nb - 2) & 1) == 1)
            def _():
                pltpu.make_async_copy(rows1, acc.at[idxb1], scat1).wait()
        @pl.when((nb > 0) & (((nb - 1) & 1) == 0))
        def _():
            pltpu.make_async_copy(rows0, acc.at[idxb0], scat0).wait()
        @pl.when((nb > 0) & (((nb - 1) & 1) == 1))
        def _():
            pltpu.make_async_copy(rows1, acc.at[idxb1], scat1).wait()
        pltpu.make_async_copy(cnt_v, cnt_sh.at[ident], souts).wait()
        plsc.subcore_barrier()

        obase = s * (P8 // NSUB)
        for j in range(7):
            pltpu.async_copy(acc.at[pl.ds(obase + j * 56, 56)],
                             outh.at[rid, pl.ds(obase + j * 56, 56)], souts)
        pltpu.async_copy(acc.at[pl.ds(obase + 392, 32)],
                         outh.at[rid, pl.ds(obase + 392, 32)], souts)
        @pl.when(s < CROWS // 8)
        def _():
            pltpu.async_copy(cnt_sh.at[pl.ds(s * 8, 8)],
                             outch.at[rid, pl.ds(s * 8, 8)], souts)
        for j in range(7):
            pltpu.make_async_copy(
                acc.at[pl.ds(obase + j * 56, 56)],
                outh.at[rid, pl.ds(obase + j * 56, 56)], souts).wait()
        pltpu.make_async_copy(acc.at[pl.ds(obase + 392, 32)],
                              outh.at[rid, pl.ds(obase + 392, 32)],
                              souts).wait()
        @pl.when(s < CROWS // 8)
        def _():
            pltpu.make_async_copy(cnt_sh.at[pl.ds(s * 8, 8)],
                                  outch.at[rid, pl.ds(s * 8, 8)],
                                  souts).wait()
        plsc.subcore_barrier()


def _sc_scatter(xa, key, src, identa):
    mesh = plsc.VectorSubcoreMesh(core_axis_name="c", subcore_axis_name="s")
    return pl.kernel(
        _sc_body,
        mesh=mesh,
        compiler_params=pltpu.CompilerParams(needs_layout_passes=False),
        out_type=(
            jax.ShapeDtypeStruct((NRANGE, P8, C), jnp.float32),
            jax.ShapeDtypeStruct((NRANGE, CROWS, C), jnp.float32),
        ),
        scratch_types=[
            pltpu.VMEM((ECH,), jnp.int32),
            pltpu.VMEM((ECH,), jnp.int32),
            pltpu.VMEM((ECH,), jnp.int32),
            pltpu.VMEM((ECH,), jnp.int32),
            pltpu.VMEM((CAP // B, B), jnp.int32),
            pltpu.VMEM((B,), jnp.int32),
            pltpu.VMEM((B,), jnp.int32),
            pltpu.VMEM((B,), jnp.int32),
            pltpu.VMEM((B,), jnp.int32),
            pltpu.VMEM((B, C), jnp.float32),
            pltpu.VMEM((B, C), jnp.float32),
            pltpu.VMEM((ZROWS, C), jnp.float32),
            pltpu.VMEM((CROWS, C), jnp.float32),
            pltpu.VMEM((CROWS,), jnp.int32),
            pltpu.VMEM_SHARED((ACC_ROWS, C), jnp.float32),
            pltpu.VMEM_SHARED((CROWS, C), jnp.float32),
            pltpu.SemaphoreType.DMA,
            pltpu.SemaphoreType.DMA,
            pltpu.SemaphoreType.DMA,
            pltpu.SemaphoreType.DMA,
            pltpu.SemaphoreType.DMA,
            pltpu.SemaphoreType.DMA,
            pltpu.SemaphoreType.DMA,
        ],
    )(xa, key, src, identa)


def _tc_body(x_ref, s_ref, c_ref, w_ref, root_ref, bias_ref, o_ref):
    acc = jnp.dot(x_ref[...], root_ref[...],
                  preferred_element_type=jnp.float32) + bias_ref[...]
    cnt = jnp.maximum(c_ref[...], 1.0)
    for t in range(R):
        mean = s_ref[:, t, :] / cnt[:, t:t + 1]
        acc = acc + jnp.dot(mean, w_ref[t], preferred_element_type=jnp.float32)
    o_ref[...] = jnp.maximum(acc, 0.0)


def _tc_combine(x, s3, cnt2, weight, root, bias2):
    BN = 400
    return pl.pallas_call(
        _tc_body,
        grid=(N // BN,),
        in_specs=[
            pl.BlockSpec((BN, C), lambda i: (i, 0)),
            pl.BlockSpec((BN, R, C), lambda i: (i, 0, 0)),
            pl.BlockSpec((BN, R), lambda i: (i, 0)),
            pl.BlockSpec((R, C, C), lambda i: (0, 0, 0)),
            pl.BlockSpec((C, C), lambda i: (0, 0)),
            pl.BlockSpec((1, C), lambda i: (0, 0)),
        ],
        out_specs=pl.BlockSpec((BN, C), lambda i: (i, 0)),
        out_shape=jax.ShapeDtypeStruct((N, C), jnp.float32),
    )(x, s3, cnt2, weight, root, bias2)


@jax.jit
def kernel(node_features, edge_index, edge_type, weight, root, bias):
    x = node_features.astype(jnp.float32)
    src = edge_index[0].astype(jnp.int32)
    dst = edge_index[1].astype(jnp.int32)
    et = edge_type.astype(jnp.int32)
    key = dst * 8 + et
    identa = jnp.arange(CROWS, dtype=jnp.int32)
    sums, cnts = _sc_scatter(x, key, src, identa)
    s3 = sums.reshape(NRANGE * P, R, C)
    cnt2 = cnts.reshape(NRANGE, CROWS * C)[:, :P8].reshape(NRANGE * P, R)
    return _tc_combine(x, s3, cnt2, weight, root.astype(jnp.float32),
                       bias.reshape(1, C).astype(jnp.float32))


# speedup vs baseline: 11.8585x; 1.0007x over previous
import jax
import jax.numpy as jnp
from jax import lax
from jax.experimental import pallas as pl
from jax.experimental.pallas import tpu as pltpu
from jax.experimental.pallas import tpu_sc as plsc

N = 10000
E = 320000
C = 128
R = 8
P = 848
P8 = P * 8
NRANGE = 12
NPASS = 6
NSUB = 16
EPT = E // NSUB
ECH = 2000
NCH = EPT // ECH
CAP = 20480
B = 128
ACC_ROWS = 6912
DUMMY = P8
ZROWS = 32
P8DUP = 8192
CROWS = 56


def _sc_body(xa, keyh, srch, identh, outh, outch,
             key_c0, key_c1, src_c0, src_c1, cpack, idxb0, idxb1,
             srcb0, srcb1, rows0, rows1, zbuf, cnt_v, ident, acc, cnt_sh,
             sem0, sem1, scat0, scat1, szero, schunk, souts):
    c = lax.axis_index("c")
    s = lax.axis_index("s")

    pltpu.sync_copy(identh, ident)

    zero16f = jnp.zeros((16,), jnp.float32)
    for zr in range(ZROWS):
        for zc in range(C // 16):
            zbuf[zr, pl.ds(zc * 16, 16)] = zero16f

    iota = lax.iota(jnp.int32, 16)
    dummyv = jnp.full((16,), DUMMY, jnp.int32)
    zerov = jnp.zeros((16,), jnp.int32)
    ones16f = jnp.ones((16,), jnp.float32)

    for pass_ in range(NPASS):
        rid = pass_ * 2 + c
        lo8 = rid * P8

        zbase = s * (ACC_ROWS // NSUB)
        for j in range(13):
            pltpu.async_copy(zbuf, acc.at[pl.ds(zbase + j * ZROWS, ZROWS)],
                             szero)
        pltpu.async_copy(zbuf.at[pl.ds(0, 16)],
                         acc.at[pl.ds(zbase + 416, 16)], szero)
        @pl.when(s < CROWS // 8)
        def _():
            pltpu.async_copy(zbuf.at[pl.ds(0, 8)], cnt_sh.at[pl.ds(s * 8, 8)],
                             szero)

        lo8v = jnp.full((16,), lo8, jnp.int32)
        ebufs = ((key_c0, src_c0), (key_c1, src_c1))

        def fire_chunk(ch, bsel):
            ebase = s * EPT + ch * ECH
            pltpu.async_copy(keyh.at[pl.ds(ebase, ECH)], ebufs[bsel][0], schunk)
            pltpu.async_copy(srch.at[pl.ds(ebase, ECH)], ebufs[bsel][1], schunk)

        def wait_chunk(ch, bsel):
            ebase = s * EPT + ch * ECH
            pltpu.make_async_copy(keyh.at[pl.ds(ebase, ECH)], ebufs[bsel][0],
                                  schunk).wait()
            pltpu.make_async_copy(srch.at[pl.ds(ebase, ECH)], ebufs[bsel][1],
                                  schunk).wait()

        fire_chunk(0, 0)

        def zc_body(j, _):
            for zc in range(C // 16):
                cnt_v[j, pl.ds(zc * 16, 16)] = zero16f
            return 0
        lax.fori_loop(0, CROWS, zc_body, 0)

        for j in range(13):
            pltpu.make_async_copy(
                zbuf, acc.at[pl.ds(zbase + j * ZROWS, ZROWS)], szero).wait()
        pltpu.make_async_copy(zbuf.at[pl.ds(0, 16)],
                              acc.at[pl.ds(zbase + 416, 16)], szero).wait()
        @pl.when(s < CROWS // 8)
        def _():
            pltpu.make_async_copy(zbuf.at[pl.ds(0, 8)],
                                  cnt_sh.at[pl.ds(s * 8, 8)], szero).wait()
        plsc.subcore_barrier()

        def unpack(j, idxb, srcb):
            jb = j * B
            for cc in range(C // 16):
                v = cpack[pl.ds(jb + cc * 16, 16)]
                srcb[pl.ds(cc * 16, 16)] = lax.shift_right_logical(v, 13)
                idxb[pl.ds(cc * 16, 16)] = v & (P8DUP - 1)

        def step_impl(j, idxb, srcb, rows, sem, scat,
                      idxo, srco, rowso, semo, scato):
            @pl.when(j >= 2)
            def _():
                pltpu.make_async_copy(rows, acc.at[idxb], scat).wait()
            unpack(j, idxb, srcb)
            pltpu.async_copy(xa.at[srcb], rows, sem)
            @pl.when(j >= 1)
            def _():
                pltpu.make_async_copy(xa.at[srco], rowso, semo).wait()
                pltpu.async_copy(rowso, acc.at[idxo], scato, add=True)

        def issue_step(j, carry):
            @pl.when((j & 1) == 0)
            def _():
                step_impl(j, idxb0, srcb0, rows0, sem0, scat0,
                          idxb1, srcb1, rows1, sem1, scat1)
            @pl.when((j & 1) == 1)
            def _():
                step_impl(j, idxb1, srcb1, rows1, sem1, scat1,
                          idxb0, srcb0, rows0, sem0, scat0)
            return carry

        def make_scan_body(key_c, src_c):
            def scan_body(i, off):
                for u in range(5):
                    k = key_c[pl.ds(i * 80 + u * 16, 16)]
                    kl = k - lo8v
                    m = kl.astype(jnp.uint32) < jnp.uint32(P8)
                    pos = off + plsc.cumsum(m.astype(jnp.int32)) - 1
                    sv = src_c[pl.ds(i * 80 + u * 16, 16)]
                    plsc.store_scatter(cpack, [pos],
                                       lax.shift_left(sv, 13) | kl, mask=m)
                    plsc.addupdate_scatter(
                        cnt_v, [lax.shift_right_logical(kl, 7), kl & (C - 1)],
                        ones16f, mask=m)
                    off = off + plsc.all_reduce_population_count(m)
                return off
            return scan_body

        def chunks_body(co, carry):
            off_c, done = carry
            for bsel in range(2):
                ch = co * 2 + bsel
                wait_chunk(ch, bsel)
                @pl.when(ch + 1 < NCH)
                def _():
                    fire_chunk(ch + 1, 1 - bsel)
                off_c = lax.fori_loop(
                    0, ECH // 80, make_scan_body(*ebufs[bsel]), off_c)
                ready = lax.shift_right_logical(jnp.max(off_c), 7)
                lax.fori_loop(done, ready, issue_step, 0)
                done = ready
            return (off_c, done)
        off_v, done = lax.fori_loop(
            0, NCH // 2, chunks_body,
            (jnp.zeros((16,), jnp.int32), jnp.int32(0)))
        off = jnp.max(off_v)

        pltpu.async_copy(cnt_v, cnt_sh.at[ident], souts, add=True)

        for j in range(B // 16):
            pos = off_v + (j * 16) + iota
            plsc.store_scatter(cpack, [pos], dummyv)
        nb = lax.shift_right_logical(off + (B - 1), 7)
        lax.fori_loop(done, nb, issue_step, 0)
        @pl.when((nb > 0) & (((nb - 1) & 1) == 0))
        def _():
            pltpu.make_async_copy(xa.at[srcb0], rows0, sem0).wait()
            pltpu.async_copy(rows0, acc.at[idxb0], scat0, add=True)
        @pl.when((nb > 0) & (((nb - 1) & 1) == 1))
        def _():
            pltpu.make_async_copy(xa.at[srcb1], rows1, sem1).wait()
            pltpu.async_copy(rows1, acc.at[idxb1], scat1, add=True)
        @pl.when(nb >= 2)
        def _():
            @pl.when(((nb - 2) & 1) == 0)
            def _():
                pltpu.make_async_copy(rows0, acc.at[idxb0], scat0).wait()
            @pl.when(((nb - 2) & 1) == 1)
            def _():
                pltpu.make_async_copy(rows1, acc.at[idxb1], scat1).wait()
        @pl.when((nb > 0) & (((nb - 1) & 1) == 0))
        def _():
            pltpu.make_async_copy(rows0, acc.at[idxb0], scat0).wait()
        @pl.when((nb > 0) & (((nb - 1) & 1) == 1))
        def _():
            pltpu.make_async_copy(rows1, acc.at[idxb1], scat1).wait()
        pltpu.make_async_copy(cnt_v, cnt_sh.at[ident], souts).wait()
        plsc.subcore_barrier()

        obase = s * (P8 // NSUB)
        for j in range(7):
            pltpu.async_copy(acc.at[pl.ds(obase + j * 56, 56)],
                             outh.at[rid, pl.ds(obase + j * 56, 56)], souts)
        pltpu.async_copy(acc.at[pl.ds(obase + 392, 32)],
                         outh.at[rid, pl.ds(obase + 392, 32)], souts)
        @pl.when(s < CROWS // 8)
        def _():
            pltpu.async_copy(cnt_sh.at[pl.ds(s * 8, 8)],
                             outch.at[rid, pl.ds(s * 8, 8)], souts)
        for j in range(7):
            pltpu.make_async_copy(
                acc.at[pl.ds(obase + j * 56, 56)],
                outh.at[rid, pl.ds(obase + j * 56, 56)], souts).wait()
        pltpu.make_async_copy(acc.at[pl.ds(obase + 392, 32)],
                              outh.at[rid, pl.ds(obase + 392, 32)],
                              souts).wait()
        @pl.when(s < CROWS // 8)
        def _():
            pltpu.make_async_copy(cnt_sh.at[pl.ds(s * 8, 8)],
                                  outch.at[rid, pl.ds(s * 8, 8)],
                                  souts).wait()
        plsc.subcore_barrier()


def _sc_scatter(xa, key, src, identa):
    mesh = plsc.VectorSubcoreMesh(core_axis_name="c", subcore_axis_name="s")
    return pl.kernel(
        _sc_body,
        mesh=mesh,
        compiler_params=pltpu.CompilerParams(needs_layout_passes=False),
        out_type=(
            jax.ShapeDtypeStruct((NRANGE, P8, C), jnp.float32),
            jax.ShapeDtypeStruct((NRANGE, CROWS, C), jnp.float32),
        ),
        scratch_types=[
            pltpu.VMEM((ECH,), jnp.int32),
            pltpu.VMEM((ECH,), jnp.int32),
            pltpu.VMEM((ECH,), jnp.int32),
            pltpu.VMEM((ECH,), jnp.int32),
            pltpu.VMEM((CAP,), jnp.int32),
            pltpu.VMEM((B,), jnp.int32),
            pltpu.VMEM((B,), jnp.int32),
            pltpu.VMEM((B,), jnp.int32),
            pltpu.VMEM((B,), jnp.int32),
            pltpu.VMEM((B, C), jnp.float32),
            pltpu.VMEM((B, C), jnp.float32),
            pltpu.VMEM((ZROWS, C), jnp.float32),
            pltpu.VMEM((CROWS, C), jnp.float32),
            pltpu.VMEM((CROWS,), jnp.int32),
            pltpu.VMEM_SHARED((ACC_ROWS, C), jnp.float32),
            pltpu.VMEM_SHARED((CROWS, C), jnp.float32),
            pltpu.SemaphoreType.DMA,
            pltpu.SemaphoreType.DMA,
            pltpu.SemaphoreType.DMA,
            pltpu.SemaphoreType.DMA,
            pltpu.SemaphoreType.DMA,
            pltpu.SemaphoreType.DMA,
            pltpu.SemaphoreType.DMA,
        ],
    )(xa, key, src, identa)


def _tc_body(x_ref, s_ref, c_ref, w_ref, root_ref, bias_ref, o_ref):
    acc = jnp.dot(x_ref[...], root_ref[...],
                  preferred_element_type=jnp.float32) + bias_ref[...]
    cnt = jnp.maximum(c_ref[...], 1.0)
    for t in range(R):
        mean = s_ref[:, t, :] / cnt[:, t:t + 1]
        acc = acc + jnp.dot(mean, w_ref[t], preferred_element_type=jnp.float32)
    o_ref[...] = jnp.maximum(acc, 0.0)


def _tc_combine(x, s3, cnt2, weight, root, bias2):
    BN = 400
    return pl.pallas_call(
        _tc_body,
        grid=(N // BN,),
        in_specs=[
            pl.BlockSpec((BN, C), lambda i: (i, 0)),
            pl.BlockSpec((BN, R, C), lambda i: (i, 0, 0)),
            pl.BlockSpec((BN, R), lambda i: (i, 0)),
            pl.BlockSpec((R, C, C), lambda i: (0, 0, 0)),
            pl.BlockSpec((C, C), lambda i: (0, 0)),
            pl.BlockSpec((1, C), lambda i: (0, 0)),
        ],
        out_specs=pl.BlockSpec((BN, C), lambda i: (i, 0)),
        out_shape=jax.ShapeDtypeStruct((N, C), jnp.float32),
    )(x, s3, cnt2, weight, root, bias2)


@jax.jit
def kernel(node_features, edge_index, edge_type, weight, root, bias):
    x = node_features.astype(jnp.float32)
    src = edge_index[0].astype(jnp.int32)
    dst = edge_index[1].astype(jnp.int32)
    et = edge_type.astype(jnp.int32)
    key = dst * 8 + et
    identa = jnp.arange(CROWS, dtype=jnp.int32)
    sums, cnts = _sc_scatter(x, key, src, identa)
    s3 = sums.reshape(NRANGE * P, R, C)
    cnt2 = cnts.reshape(NRANGE, CROWS * C)[:, :P8].reshape(NRANGE * P, R)
    return _tc_combine(x, s3, cnt2, weight, root.astype(jnp.float32),
                       bias.reshape(1, C).astype(jnp.float32))


# speedup vs baseline: 17.6947x; 1.4922x over previous
import jax
import jax.numpy as jnp
from jax import lax
from jax.experimental import pallas as pl
from jax.experimental.pallas import tpu as pltpu
from jax.experimental.pallas import tpu_sc as plsc

N = 10000
E = 320000
C = 128
R = 8
P = 848
P8 = P * 8
NRANGE = 12
NPASS = 6
NSUB = 16
EPT = E // NSUB
ECH = 2000
NCH = EPT // ECH
CAP = 20480
B = 64
BSH = 6
NBUF = 4
ACC_ROWS = 6912
DUMMY = P8
ZROWS = 32
P8DUP = 8192
CROWS = 56


def _sc_body(xa, keyh, srch, identh, outh, outch,
             key_c0, key_c1, src_c0, src_c1, cpack,
             idxb0, idxb1, idxb2, idxb3, srcb0, srcb1, srcb2, srcb3,
             rows0, rows1, rows2, rows3, zbuf, cnt_v, ident, acc, cnt_sh,
             sem0, sem1, sem2, sem3, scat0, scat1, scat2, scat3,
             szero, schunk, souts):
    c = lax.axis_index("c")
    s = lax.axis_index("s")

    pltpu.sync_copy(identh, ident)

    zero16f = jnp.zeros((16,), jnp.float32)
    for zr in range(ZROWS):
        for zc in range(C // 16):
            zbuf[zr, pl.ds(zc * 16, 16)] = zero16f

    iota = lax.iota(jnp.int32, 16)
    dummyv = jnp.full((16,), DUMMY, jnp.int32)
    zerov = jnp.zeros((16,), jnp.int32)
    ones16f = jnp.ones((16,), jnp.float32)

    for pass_ in range(NPASS):
        rid = pass_ * 2 + c
        lo8 = rid * P8

        zbase = s * (ACC_ROWS // NSUB)
        for j in range(13):
            pltpu.async_copy(zbuf, acc.at[pl.ds(zbase + j * ZROWS, ZROWS)],
                             szero)
        pltpu.async_copy(zbuf.at[pl.ds(0, 16)],
                         acc.at[pl.ds(zbase + 416, 16)], szero)
        @pl.when(s < CROWS // 8)
        def _():
            pltpu.async_copy(zbuf.at[pl.ds(0, 8)], cnt_sh.at[pl.ds(s * 8, 8)],
                             szero)

        lo8v = jnp.full((16,), lo8, jnp.int32)
        ebufs = ((key_c0, src_c0), (key_c1, src_c1))

        def fire_chunk(ch, bsel):
            ebase = s * EPT + ch * ECH
            pltpu.async_copy(keyh.at[pl.ds(ebase, ECH)], ebufs[bsel][0], schunk)
            pltpu.async_copy(srch.at[pl.ds(ebase, ECH)], ebufs[bsel][1], schunk)

        def wait_chunk(ch, bsel):
            ebase = s * EPT + ch * ECH
            pltpu.make_async_copy(keyh.at[pl.ds(ebase, ECH)], ebufs[bsel][0],
                                  schunk).wait()
            pltpu.make_async_copy(srch.at[pl.ds(ebase, ECH)], ebufs[bsel][1],
                                  schunk).wait()

        fire_chunk(0, 0)

        def zc_body(j, _):
            for zc in range(C // 16):
                cnt_v[j, pl.ds(zc * 16, 16)] = zero16f
            return 0
        lax.fori_loop(0, CROWS, zc_body, 0)

        for j in range(13):
            pltpu.make_async_copy(
                zbuf, acc.at[pl.ds(zbase + j * ZROWS, ZROWS)], szero).wait()
        pltpu.make_async_copy(zbuf.at[pl.ds(0, 16)],
                              acc.at[pl.ds(zbase + 416, 16)], szero).wait()
        @pl.when(s < CROWS // 8)
        def _():
            pltpu.make_async_copy(zbuf.at[pl.ds(0, 8)],
                                  cnt_sh.at[pl.ds(s * 8, 8)], szero).wait()
        plsc.subcore_barrier()

        bufs = ((idxb0, srcb0, rows0, sem0, scat0),
                (idxb1, srcb1, rows1, sem1, scat1),
                (idxb2, srcb2, rows2, sem2, scat2),
                (idxb3, srcb3, rows3, sem3, scat3))

        def unpack(j, idxb, srcb):
            jb = j * B
            for cc in range(B // 16):
                v = cpack[pl.ds(jb + cc * 16, 16)]
                srcb[pl.ds(cc * 16, 16)] = lax.shift_right_logical(v, 13)
                idxb[pl.ds(cc * 16, 16)] = v & (P8DUP - 1)

        def step_impl(j, cur, com):
            idxb, srcb, rows, sem, scat = cur
            idxc, srcc, rowsc, semc, scatc = com
            @pl.when(j >= NBUF)
            def _():
                pltpu.make_async_copy(rows, acc.at[idxb], scat).wait()
            unpack(j, idxb, srcb)
            pltpu.async_copy(xa.at[srcb], rows, sem)
            @pl.when(j >= 2)
            def _():
                pltpu.make_async_copy(xa.at[srcc], rowsc, semc).wait()
                pltpu.async_copy(rowsc, acc.at[idxc], scatc, add=True)

        def issue_step(j, carry):
            for b in range(NBUF):
                @pl.when((j & (NBUF - 1)) == b)
                def _(b=b):
                    step_impl(j, bufs[b], bufs[(b + 2) & (NBUF - 1)])
            return carry

        def make_scan_body(key_c, src_c):
            def scan_body(i, off):
                for u in range(5):
                    k = key_c[pl.ds(i * 80 + u * 16, 16)]
                    kl = k - lo8v
                    m = kl.astype(jnp.uint32) < jnp.uint32(P8)
                    pos = off + plsc.cumsum(m.astype(jnp.int32)) - 1
                    sv = src_c[pl.ds(i * 80 + u * 16, 16)]
                    plsc.store_scatter(cpack, [pos],
                                       lax.shift_left(sv, 13) | kl, mask=m)
                    plsc.addupdate_scatter(
                        cnt_v, [lax.shift_right_logical(kl, 7), kl & (C - 1)],
                        ones16f, mask=m)
                    off = off + plsc.all_reduce_population_count(m)
                return off
            return scan_body

        def chunks_body(co, carry):
            off_c, done = carry
            for bsel in range(2):
                ch = co * 2 + bsel
                wait_chunk(ch, bsel)
                @pl.when(ch + 1 < NCH)
                def _():
                    fire_chunk(ch + 1, 1 - bsel)
                off_c = lax.fori_loop(
                    0, ECH // 80, make_scan_body(*ebufs[bsel]), off_c)
                ready = lax.shift_right_logical(jnp.max(off_c), BSH)
                lax.fori_loop(done, ready, issue_step, 0)
                done = ready
            return (off_c, done)
        off_v, done = lax.fori_loop(
            0, NCH // 2, chunks_body,
            (jnp.zeros((16,), jnp.int32), jnp.int32(0)))
        off = jnp.max(off_v)

        pltpu.async_copy(cnt_v, cnt_sh.at[ident], souts, add=True)

        for j in range(B // 16):
            pos = off_v + (j * 16) + iota
            plsc.store_scatter(cpack, [pos], dummyv)
        nb = lax.shift_right_logical(off + (B - 1), BSH)
        lax.fori_loop(done, nb, issue_step, 0)
        for t_off in (2, 1):
            @pl.when(nb >= t_off)
            def _(t_off=t_off):
                for b in range(NBUF):
                    @pl.when(((nb - t_off) & (NBUF - 1)) == b)
                    def _(b=b):
                        idxb, srcb, rows, sem, scat = bufs[b]
                        pltpu.make_async_copy(xa.at[srcb], rows, sem).wait()
                        pltpu.async_copy(rows, acc.at[idxb], scat, add=True)
        for b in range(NBUF):
            @pl.when(nb >= b + 1)
            def _(b=b):
                idxb, srcb, rows, sem, scat = bufs[b]
                pltpu.make_async_copy(rows, acc.at[idxb], scat).wait()
        pltpu.make_async_copy(cnt_v, cnt_sh.at[ident], souts).wait()
        plsc.subcore_barrier()

        obase = s * (P8 // NSUB)
        for j in range(7):
            pltpu.async_copy(acc.at[pl.ds(obase + j * 56, 56)],
                             outh.at[rid, pl.ds(obase + j * 56, 56)], souts)
        pltpu.async_copy(acc.at[pl.ds(obase + 392, 32)],
                         outh.at[rid, pl.ds(obase + 392, 32)], souts)
        @pl.when(s < CROWS // 8)
        def _():
            pltpu.async_copy(cnt_sh.at[pl.ds(s * 8, 8)],
                             outch.at[rid, pl.ds(s * 8, 8)], souts)
        for j in range(7):
            pltpu.make_async_copy(
                acc.at[pl.ds(obase + j * 56, 56)],
                outh.at[rid, pl.ds(obase + j * 56, 56)], souts).wait()
        pltpu.make_async_copy(acc.at[pl.ds(obase + 392, 32)],
                              outh.at[rid, pl.ds(obase + 392, 32)],
                              souts).wait()
        @pl.when(s < CROWS // 8)
        def _():
            pltpu.make_async_copy(cnt_sh.at[pl.ds(s * 8, 8)],
                                  outch.at[rid, pl.ds(s * 8, 8)],
                                  souts).wait()
        plsc.subcore_barrier()


def _sc_scatter(xa, key, src, identa):
    mesh = plsc.VectorSubcoreMesh(core_axis_name="c", subcore_axis_name="s")
    return pl.kernel(
        _sc_body,
        mesh=mesh,
        compiler_params=pltpu.CompilerParams(needs_layout_passes=False),
        out_type=(
            jax.ShapeDtypeStruct((NRANGE, P8, C), jnp.float32),
            jax.ShapeDtypeStruct((NRANGE, CROWS, C), jnp.float32),
        ),
        scratch_types=[
            pltpu.VMEM((ECH,), jnp.int32),
            pltpu.VMEM((ECH,), jnp.int32),
            pltpu.VMEM((ECH,), jnp.int32),
            pltpu.VMEM((ECH,), jnp.int32),
            pltpu.VMEM((CAP,), jnp.int32),
            pltpu.VMEM((B,), jnp.int32),
            pltpu.VMEM((B,), jnp.int32),
            pltpu.VMEM((B,), jnp.int32),
            pltpu.VMEM((B,), jnp.int32),
            pltpu.VMEM((B,), jnp.int32),
            pltpu.VMEM((B,), jnp.int32),
            pltpu.VMEM((B,), jnp.int32),
            pltpu.VMEM((B,), jnp.int32),
            pltpu.VMEM((B, C), jnp.float32),
            pltpu.VMEM((B, C), jnp.float32),
            pltpu.VMEM((B, C), jnp.float32),
            pltpu.VMEM((B, C), jnp.float32),
            pltpu.VMEM((ZROWS, C), jnp.float32),
            pltpu.VMEM((CROWS, C), jnp.float32),
            pltpu.VMEM((CROWS,), jnp.int32),
            pltpu.VMEM_SHARED((ACC_ROWS, C), jnp.float32),
            pltpu.VMEM_SHARED((CROWS, C), jnp.float32),
            pltpu.SemaphoreType.DMA,
            pltpu.SemaphoreType.DMA,
            pltpu.SemaphoreType.DMA,
            pltpu.SemaphoreType.DMA,
            pltpu.SemaphoreType.DMA,
            pltpu.SemaphoreType.DMA,
            pltpu.SemaphoreType.DMA,
            pltpu.SemaphoreType.DMA,
            pltpu.SemaphoreType.DMA,
            pltpu.SemaphoreType.DMA,
            pltpu.SemaphoreType.DMA,
        ],
    )(xa, key, src, identa)


def _tc_body(x_ref, s_ref, c_ref, w_ref, root_ref, bias_ref, o_ref):
    acc = jnp.dot(x_ref[...], root_ref[...],
                  preferred_element_type=jnp.float32) + bias_ref[...]
    cnt = jnp.maximum(c_ref[...], 1.0)
    for t in range(R):
        mean = s_ref[:, t, :] / cnt[:, t:t + 1]
        acc = acc + jnp.dot(mean, w_ref[t], preferred_element_type=jnp.float32)
    o_ref[...] = jnp.maximum(acc, 0.0)


def _tc_combine(x, s3, cnt2, weight, root, bias2):
    BN = 400
    return pl.pallas_call(
        _tc_body,
        grid=(N // BN,),
        in_specs=[
            pl.BlockSpec((BN, C), lambda i: (i, 0)),
            pl.BlockSpec((BN, R, C), lambda i: (i, 0, 0)),
            pl.BlockSpec((BN, R), lambda i: (i, 0)),
            pl.BlockSpec((R, C, C), lambda i: (0, 0, 0)),
            pl.BlockSpec((C, C), lambda i: (0, 0)),
            pl.BlockSpec((1, C), lambda i: (0, 0)),
        ],
        out_specs=pl.BlockSpec((BN, C), lambda i: (i, 0)),
        out_shape=jax.ShapeDtypeStruct((N, C), jnp.float32),
    )(x, s3, cnt2, weight, root, bias2)


@jax.jit
def kernel(node_features, edge_index, edge_type, weight, root, bias):
    x = node_features.astype(jnp.float32)
    src = edge_index[0].astype(jnp.int32)
    dst = edge_index[1].astype(jnp.int32)
    et = edge_type.astype(jnp.int32)
    key = dst * 8 + et
    identa = jnp.arange(CROWS, dtype=jnp.int32)
    sums, cnts = _sc_scatter(x, key, src, identa)
    s3 = sums.reshape(NRANGE * P, R, C)
    cnt2 = cnts.reshape(NRANGE, CROWS * C)[:, :P8].reshape(NRANGE * P, R)
    return _tc_combine(x, s3, cnt2, weight, root.astype(jnp.float32),
                       bias.reshape(1, C).astype(jnp.float32))


# speedup vs baseline: 21.9475x; 1.2403x over previous
import jax
import jax.numpy as jnp
from jax import lax
from jax.experimental import pallas as pl
from jax.experimental.pallas import tpu as pltpu
from jax.experimental.pallas import tpu_sc as plsc

N = 10000
E = 320000
C = 128
R = 8
P = 848
P8 = P * 8
NRANGE = 12
NPASS = 6
NSUB = 16
EPT = E // NSUB
ECH = 2000
NCH = EPT // ECH
CAP = 20480
B = 32
BSH = 5
NBUF = 8
LAG = 4
ACC_ROWS = 6912
DUMMY = P8
ZROWS = 32
P8DUP = 8192
CROWS = 56


def _sc_body(xa, keyh, srch, identh, outh, outch,
             key_c0, key_c1, src_c0, src_c1, cpack,
             idxb0, idxb1, idxb2, idxb3, idxb4, idxb5, idxb6, idxb7,
             srcb0, srcb1, srcb2, srcb3, srcb4, srcb5, srcb6, srcb7,
             rows0, rows1, rows2, rows3, rows4, rows5, rows6, rows7,
             zbuf, cnt_v, ident, acc, cnt_sh,
             sem0, sem1, sem2, sem3, sem4, sem5, sem6, sem7,
             scat0, scat1, scat2, scat3, scat4, scat5, scat6, scat7,
             szero, schunk, souts):
    c = lax.axis_index("c")
    s = lax.axis_index("s")

    pltpu.sync_copy(identh, ident)

    zero16f = jnp.zeros((16,), jnp.float32)
    for zr in range(ZROWS):
        for zc in range(C // 16):
            zbuf[zr, pl.ds(zc * 16, 16)] = zero16f

    iota = lax.iota(jnp.int32, 16)
    dummyv = jnp.full((16,), DUMMY, jnp.int32)
    zerov = jnp.zeros((16,), jnp.int32)
    ones16f = jnp.ones((16,), jnp.float32)

    for pass_ in range(NPASS):
        rid = pass_ * 2 + c
        lo8 = rid * P8

        zbase = s * (ACC_ROWS // NSUB)
        for j in range(13):
            pltpu.async_copy(zbuf, acc.at[pl.ds(zbase + j * ZROWS, ZROWS)],
                             szero)
        pltpu.async_copy(zbuf.at[pl.ds(0, 16)],
                         acc.at[pl.ds(zbase + 416, 16)], szero)
        @pl.when(s < CROWS // 8)
        def _():
            pltpu.async_copy(zbuf.at[pl.ds(0, 8)], cnt_sh.at[pl.ds(s * 8, 8)],
                             szero)

        lo8v = jnp.full((16,), lo8, jnp.int32)
        ebufs = ((key_c0, src_c0), (key_c1, src_c1))

        def fire_chunk(ch, bsel):
            ebase = s * EPT + ch * ECH
            pltpu.async_copy(keyh.at[pl.ds(ebase, ECH)], ebufs[bsel][0], schunk)
            pltpu.async_copy(srch.at[pl.ds(ebase, ECH)], ebufs[bsel][1], schunk)

        def wait_chunk(ch, bsel):
            ebase = s * EPT + ch * ECH
            pltpu.make_async_copy(keyh.at[pl.ds(ebase, ECH)], ebufs[bsel][0],
                                  schunk).wait()
            pltpu.make_async_copy(srch.at[pl.ds(ebase, ECH)], ebufs[bsel][1],
                                  schunk).wait()

        fire_chunk(0, 0)

        def zc_body(j, _):
            for zc in range(C // 16):
                cnt_v[j, pl.ds(zc * 16, 16)] = zero16f
            return 0
        lax.fori_loop(0, CROWS, zc_body, 0)

        for j in range(13):
            pltpu.make_async_copy(
                zbuf, acc.at[pl.ds(zbase + j * ZROWS, ZROWS)], szero).wait()
        pltpu.make_async_copy(zbuf.at[pl.ds(0, 16)],
                              acc.at[pl.ds(zbase + 416, 16)], szero).wait()
        @pl.when(s < CROWS // 8)
        def _():
            pltpu.make_async_copy(zbuf.at[pl.ds(0, 8)],
                                  cnt_sh.at[pl.ds(s * 8, 8)], szero).wait()
        plsc.subcore_barrier()

        bufs = ((idxb0, srcb0, rows0, sem0, scat0),
                (idxb1, srcb1, rows1, sem1, scat1),
                (idxb2, srcb2, rows2, sem2, scat2),
                (idxb3, srcb3, rows3, sem3, scat3),
                (idxb4, srcb4, rows4, sem4, scat4),
                (idxb5, srcb5, rows5, sem5, scat5),
                (idxb6, srcb6, rows6, sem6, scat6),
                (idxb7, srcb7, rows7, sem7, scat7))

        def unpack(j, idxb, srcb):
            jb = j * B
            for cc in range(B // 16):
                v = cpack[pl.ds(jb + cc * 16, 16)]
                srcb[pl.ds(cc * 16, 16)] = lax.shift_right_logical(v, 13)
                idxb[pl.ds(cc * 16, 16)] = v & (P8DUP - 1)

        def step_impl(j, cur, com):
            idxb, srcb, rows, sem, scat = cur
            idxc, srcc, rowsc, semc, scatc = com
            @pl.when(j >= NBUF)
            def _():
                pltpu.make_async_copy(rows, acc.at[idxb], scat).wait()
            unpack(j, idxb, srcb)
            pltpu.async_copy(xa.at[srcb], rows, sem)
            @pl.when(j >= LAG)
            def _():
                pltpu.make_async_copy(xa.at[srcc], rowsc, semc).wait()
                pltpu.async_copy(rowsc, acc.at[idxc], scatc, add=True)

        def issue_step(j, carry):
            for b in range(NBUF):
                @pl.when((j & (NBUF - 1)) == b)
                def _(b=b):
                    step_impl(j, bufs[b], bufs[(b - LAG) & (NBUF - 1)])
            return carry

        def make_scan_body(key_c, src_c):
            def scan_body(i, off):
                for u in range(5):
                    k = key_c[pl.ds(i * 80 + u * 16, 16)]
                    kl = k - lo8v
                    m = kl.astype(jnp.uint32) < jnp.uint32(P8)
                    pos = off + plsc.cumsum(m.astype(jnp.int32)) - 1
                    sv = src_c[pl.ds(i * 80 + u * 16, 16)]
                    plsc.store_scatter(cpack, [pos],
                                       lax.shift_left(sv, 13) | kl, mask=m)
                    plsc.addupdate_scatter(
                        cnt_v, [lax.shift_right_logical(kl, 7), kl & (C - 1)],
                        ones16f, mask=m)
                    off = off + plsc.all_reduce_population_count(m)
                return off
            return scan_body

        def chunks_body(co, carry):
            off_c, done = carry
            for bsel in range(2):
                ch = co * 2 + bsel
                wait_chunk(ch, bsel)
                @pl.when(ch + 1 < NCH)
                def _():
                    fire_chunk(ch + 1, 1 - bsel)
                off_c = lax.fori_loop(
                    0, ECH // 80, make_scan_body(*ebufs[bsel]), off_c)
                ready = lax.shift_right_logical(jnp.max(off_c), BSH)
                lax.fori_loop(done, ready, issue_step, 0)
                done = ready
            return (off_c, done)
        off_v, done = lax.fori_loop(
            0, NCH // 2, chunks_body,
            (jnp.zeros((16,), jnp.int32), jnp.int32(0)))
        off = jnp.max(off_v)

        pltpu.async_copy(cnt_v, cnt_sh.at[ident], souts, add=True)

        for j in range(B // 16):
            pos = off_v + (j * 16) + iota
            plsc.store_scatter(cpack, [pos], dummyv)
        nb = lax.shift_right_logical(off + (B - 1), BSH)
        lax.fori_loop(done, nb, issue_step, 0)
        for t_off in range(LAG, 0, -1):
            @pl.when(nb >= t_off)
            def _(t_off=t_off):
                for b in range(NBUF):
                    @pl.when(((nb - t_off) & (NBUF - 1)) == b)
                    def _(b=b):
                        idxb, srcb, rows, sem, scat = bufs[b]
                        pltpu.make_async_copy(xa.at[srcb], rows, sem).wait()
                        pltpu.async_copy(rows, acc.at[idxb], scat, add=True)
        for b in range(NBUF):
            @pl.when(nb >= b + 1)
            def _(b=b):
                idxb, srcb, rows, sem, scat = bufs[b]
                pltpu.make_async_copy(rows, acc.at[idxb], scat).wait()
        pltpu.make_async_copy(cnt_v, cnt_sh.at[ident], souts).wait()
        plsc.subcore_barrier()

        obase = s * (P8 // NSUB)
        for j in range(7):
            pltpu.async_copy(acc.at[pl.ds(obase + j * 56, 56)],
                             outh.at[rid, pl.ds(obase + j * 56, 56)], souts)
        pltpu.async_copy(acc.at[pl.ds(obase + 392, 32)],
                         outh.at[rid, pl.ds(obase + 392, 32)], souts)
        @pl.when(s < CROWS // 8)
        def _():
            pltpu.async_copy(cnt_sh.at[pl.ds(s * 8, 8)],
                             outch.at[rid, pl.ds(s * 8, 8)], souts)
        for j in range(7):
            pltpu.make_async_copy(
                acc.at[pl.ds(obase + j * 56, 56)],
                outh.at[rid, pl.ds(obase + j * 56, 56)], souts).wait()
        pltpu.make_async_copy(acc.at[pl.ds(obase + 392, 32)],
                              outh.at[rid, pl.ds(obase + 392, 32)],
                              souts).wait()
        @pl.when(s < CROWS // 8)
        def _():
            pltpu.make_async_copy(cnt_sh.at[pl.ds(s * 8, 8)],
                                  outch.at[rid, pl.ds(s * 8, 8)],
                                  souts).wait()
        plsc.subcore_barrier()


def _sc_scatter(xa, key, src, identa):
    mesh = plsc.VectorSubcoreMesh(core_axis_name="c", subcore_axis_name="s")
    return pl.kernel(
        _sc_body,
        mesh=mesh,
        compiler_params=pltpu.CompilerParams(needs_layout_passes=False),
        out_type=(
            jax.ShapeDtypeStruct((NRANGE, P8, C), jnp.float32),
            jax.ShapeDtypeStruct((NRANGE, CROWS, C), jnp.float32),
        ),
        scratch_types=[
            pltpu.VMEM((ECH,), jnp.int32),
            pltpu.VMEM((ECH,), jnp.int32),
            pltpu.VMEM((ECH,), jnp.int32),
            pltpu.VMEM((ECH,), jnp.int32),
            pltpu.VMEM((CAP,), jnp.int32),
            pltpu.VMEM((B,), jnp.int32),
            pltpu.VMEM((B,), jnp.int32),
            pltpu.VMEM((B,), jnp.int32),
            pltpu.VMEM((B,), jnp.int32),
            pltpu.VMEM((B,), jnp.int32),
            pltpu.VMEM((B,), jnp.int32),
            pltpu.VMEM((B,), jnp.int32),
            pltpu.VMEM((B,), jnp.int32),
            pltpu.VMEM((B,), jnp.int32),
            pltpu.VMEM((B,), jnp.int32),
            pltpu.VMEM((B,), jnp.int32),
            pltpu.VMEM((B,), jnp.int32),
            pltpu.VMEM((B,), jnp.int32),
            pltpu.VMEM((B,), jnp.int32),
            pltpu.VMEM((B,), jnp.int32),
            pltpu.VMEM((B,), jnp.int32),
            pltpu.VMEM((B, C), jnp.float32),
            pltpu.VMEM((B, C), jnp.float32),
            pltpu.VMEM((B, C), jnp.float32),
            pltpu.VMEM((B, C), jnp.float32),
            pltpu.VMEM((B, C), jnp.float32),
            pltpu.VMEM((B, C), jnp.float32),
            pltpu.VMEM((B, C), jnp.float32),
            pltpu.VMEM((B, C), jnp.float32),
            pltpu.VMEM((ZROWS, C), jnp.float32),
            pltpu.VMEM((CROWS, C), jnp.float32),
            pltpu.VMEM((CROWS,), jnp.int32),
            pltpu.VMEM_SHARED((ACC_ROWS, C), jnp.float32),
            pltpu.VMEM_SHARED((CROWS, C), jnp.float32),
            pltpu.SemaphoreType.DMA,
            pltpu.SemaphoreType.DMA,
            pltpu.SemaphoreType.DMA,
            pltpu.SemaphoreType.DMA,
            pltpu.SemaphoreType.DMA,
            pltpu.SemaphoreType.DMA,
            pltpu.SemaphoreType.DMA,
            pltpu.SemaphoreType.DMA,
            pltpu.SemaphoreType.DMA,
            pltpu.SemaphoreType.DMA,
            pltpu.SemaphoreType.DMA,
            pltpu.SemaphoreType.DMA,
            pltpu.SemaphoreType.DMA,
            pltpu.SemaphoreType.DMA,
            pltpu.SemaphoreType.DMA,
            pltpu.SemaphoreType.DMA,
            pltpu.SemaphoreType.DMA,
            pltpu.SemaphoreType.DMA,
            pltpu.SemaphoreType.DMA,
        ],
    )(xa, key, src, identa)


def _tc_body(x_ref, s_ref, c_ref, w_ref, root_ref, bias_ref, o_ref):
    acc = jnp.dot(x_ref[...], root_ref[...],
                  preferred_element_type=jnp.float32) + bias_ref[...]
    cnt = jnp.maximum(c_ref[...], 1.0)
    for t in range(R):
        mean = s_ref[:, t, :] / cnt[:, t:t + 1]
        acc = acc + jnp.dot(mean, w_ref[t], preferred_element_type=jnp.float32)
    o_ref[...] = jnp.maximum(acc, 0.0)


def _tc_combine(x, s3, cnt2, weight, root, bias2):
    BN = 400
    return pl.pallas_call(
        _tc_body,
        grid=(N // BN,),
        in_specs=[
            pl.BlockSpec((BN, C), lambda i: (i, 0)),
            pl.BlockSpec((BN, R, C), lambda i: (i, 0, 0)),
            pl.BlockSpec((BN, R), lambda i: (i, 0)),
            pl.BlockSpec((R, C, C), lambda i: (0, 0, 0)),
            pl.BlockSpec((C, C), lambda i: (0, 0)),
            pl.BlockSpec((1, C), lambda i: (0, 0)),
        ],
        out_specs=pl.BlockSpec((BN, C), lambda i: (i, 0)),
        out_shape=jax.ShapeDtypeStruct((N, C), jnp.float32),
    )(x, s3, cnt2, weight, root, bias2)


@jax.jit
def kernel(node_features, edge_index, edge_type, weight, root, bias):
    x = node_features.astype(jnp.float32)
    src = edge_index[0].astype(jnp.int32)
    dst = edge_index[1].astype(jnp.int32)
    et = edge_type.astype(jnp.int32)
    key = dst * 8 + et
    identa = jnp.arange(CROWS, dtype=jnp.int32)
    sums, cnts = _sc_scatter(x, key, src, identa)
    s3 = sums.reshape(NRANGE * P, R, C)
    cnt2 = cnts.reshape(NRANGE, CROWS * C)[:, :P8].reshape(NRANGE * P, R)
    return _tc_combine(x, s3, cnt2, weight, root.astype(jnp.float32),
                       bias.reshape(1, C).astype(jnp.float32))


# speedup vs baseline: 22.5433x; 1.0272x over previous
import jax
import jax.numpy as jnp
from jax import lax
from jax.experimental import pallas as pl
from jax.experimental.pallas import tpu as pltpu
from jax.experimental.pallas import tpu_sc as plsc

N = 10000
E = 320000
C = 128
R = 8
P = 848
P8 = P * 8
NRANGE = 12
NPASS = 6
NSUB = 16
EPT = E // NSUB
ECH = 2000
NCH = EPT // ECH
CAP = 20480
B = 32
BSH = 5
NBUF = 8
LAG = 5
ACC_ROWS = 6912
DUMMY = P8
ZROWS = 32
P8DUP = 8192
CROWS = 56


def _sc_body(xa, keyh, srch, identh, outh, outch,
             key_c0, key_c1, src_c0, src_c1, cpack,
             idxb0, idxb1, idxb2, idxb3, idxb4, idxb5, idxb6, idxb7,
             srcb0, srcb1, srcb2, srcb3, srcb4, srcb5, srcb6, srcb7,
             rows0, rows1, rows2, rows3, rows4, rows5, rows6, rows7,
             zbuf, cnt_v, ident, acc, cnt_sh,
             sem0, sem1, sem2, sem3, sem4, sem5, sem6, sem7,
             scat0, scat1, scat2, scat3, scat4, scat5, scat6, scat7,
             szero, schunk, souts):
    c = lax.axis_index("c")
    s = lax.axis_index("s")

    pltpu.sync_copy(identh, ident)

    zero16f = jnp.zeros((16,), jnp.float32)
    for zr in range(ZROWS):
        for zc in range(C // 16):
            zbuf[zr, pl.ds(zc * 16, 16)] = zero16f

    iota = lax.iota(jnp.int32, 16)
    dummyv = jnp.full((16,), DUMMY, jnp.int32)
    zerov = jnp.zeros((16,), jnp.int32)
    ones16f = jnp.ones((16,), jnp.float32)

    for pass_ in range(NPASS):
        rid = pass_ * 2 + c
        lo8 = rid * P8

        zbase = s * (ACC_ROWS // NSUB)
        for j in range(13):
            pltpu.async_copy(zbuf, acc.at[pl.ds(zbase + j * ZROWS, ZROWS)],
                             szero)
        pltpu.async_copy(zbuf.at[pl.ds(0, 16)],
                         acc.at[pl.ds(zbase + 416, 16)], szero)
        @pl.when(s < CROWS // 8)
        def _():
            pltpu.async_copy(zbuf.at[pl.ds(0, 8)], cnt_sh.at[pl.ds(s * 8, 8)],
                             szero)

        lo8v = jnp.full((16,), lo8, jnp.int32)
        ebufs = ((key_c0, src_c0), (key_c1, src_c1))

        def fire_chunk(ch, bsel):
            ebase = s * EPT + ch * ECH
            pltpu.async_copy(keyh.at[pl.ds(ebase, ECH)], ebufs[bsel][0], schunk)
            pltpu.async_copy(srch.at[pl.ds(ebase, ECH)], ebufs[bsel][1], schunk)

        def wait_chunk(ch, bsel):
            ebase = s * EPT + ch * ECH
            pltpu.make_async_copy(keyh.at[pl.ds(ebase, ECH)], ebufs[bsel][0],
                                  schunk).wait()
            pltpu.make_async_copy(srch.at[pl.ds(ebase, ECH)], ebufs[bsel][1],
                                  schunk).wait()

        fire_chunk(0, 0)

        def zc_body(j, _):
            for zc in range(C // 16):
                cnt_v[j, pl.ds(zc * 16, 16)] = zero16f
            return 0
        lax.fori_loop(0, CROWS, zc_body, 0)

        for j in range(13):
            pltpu.make_async_copy(
                zbuf, acc.at[pl.ds(zbase + j * ZROWS, ZROWS)], szero).wait()
        pltpu.make_async_copy(zbuf.at[pl.ds(0, 16)],
                              acc.at[pl.ds(zbase + 416, 16)], szero).wait()
        @pl.when(s < CROWS // 8)
        def _():
            pltpu.make_async_copy(zbuf.at[pl.ds(0, 8)],
                                  cnt_sh.at[pl.ds(s * 8, 8)], szero).wait()
        plsc.subcore_barrier()

        bufs = ((idxb0, srcb0, rows0, sem0, scat0),
                (idxb1, srcb1, rows1, sem1, scat1),
                (idxb2, srcb2, rows2, sem2, scat2),
                (idxb3, srcb3, rows3, sem3, scat3),
                (idxb4, srcb4, rows4, sem4, scat4),
                (idxb5, srcb5, rows5, sem5, scat5),
                (idxb6, srcb6, rows6, sem6, scat6),
                (idxb7, srcb7, rows7, sem7, scat7))

        def unpack(j, idxb, srcb):
            jb = j * B
            for cc in range(B // 16):
                v = cpack[pl.ds(jb + cc * 16, 16)]
                srcb[pl.ds(cc * 16, 16)] = lax.shift_right_logical(v, 13)
                idxb[pl.ds(cc * 16, 16)] = v & (P8DUP - 1)

        def step_impl(j, cur, com):
            idxb, srcb, rows, sem, scat = cur
            idxc, srcc, rowsc, semc, scatc = com
            @pl.when(j >= NBUF)
            def _():
                pltpu.make_async_copy(rows, acc.at[idxb], scat).wait()
            unpack(j, idxb, srcb)
            pltpu.async_copy(xa.at[srcb], rows, sem)
            @pl.when(j >= LAG)
            def _():
                pltpu.make_async_copy(xa.at[srcc], rowsc, semc).wait()
                pltpu.async_copy(rowsc, acc.at[idxc], scatc, add=True)

        def issue_step(j, carry):
            for b in range(NBUF):
                @pl.when((j & (NBUF - 1)) == b)
                def _(b=b):
                    step_impl(j, bufs[b], bufs[(b - LAG) & (NBUF - 1)])
            return carry

        def make_scan_body(key_c, src_c):
            def scan_body(i, off):
                for u in range(5):
                    k = key_c[pl.ds(i * 80 + u * 16, 16)]
                    kl = k - lo8v
                    m = kl.astype(jnp.uint32) < jnp.uint32(P8)
                    pos = off + plsc.cumsum(m.astype(jnp.int32)) - 1
                    sv = src_c[pl.ds(i * 80 + u * 16, 16)]
                    plsc.store_scatter(cpack, [pos],
                                       lax.shift_left(sv, 13) | kl, mask=m)
                    plsc.addupdate_scatter(
                        cnt_v, [lax.shift_right_logical(kl, 7), kl & (C - 1)],
                        ones16f, mask=m)
                    off = off + plsc.all_reduce_population_count(m)
                return off
            return scan_body

        def chunks_body(co, carry):
            off_c, done = carry
            for bsel in range(2):
                ch = co * 2 + bsel
                wait_chunk(ch, bsel)
                @pl.when(ch + 1 < NCH)
                def _():
                    fire_chunk(ch + 1, 1 - bsel)
                off_c = lax.fori_loop(
                    0, ECH // 80, make_scan_body(*ebufs[bsel]), off_c)
                ready = lax.shift_right_logical(jnp.max(off_c), BSH)
                lax.fori_loop(done, ready, issue_step, 0)
                done = ready
            return (off_c, done)
        off_v, done = lax.fori_loop(
            0, NCH // 2, chunks_body,
            (jnp.zeros((16,), jnp.int32), jnp.int32(0)))
        off = jnp.max(off_v)

        pltpu.async_copy(cnt_v, cnt_sh.at[ident], souts, add=True)

        for j in range(B // 16):
            pos = off_v + (j * 16) + iota
            plsc.store_scatter(cpack, [pos], dummyv)
        nb = lax.shift_right_logical(off + (B - 1), BSH)
        lax.fori_loop(done, nb, issue_step, 0)
        for t_off in range(LAG, 0, -1):
            @pl.when(nb >= t_off)
            def _(t_off=t_off):
                for b in range(NBUF):
                    @pl.when(((nb - t_off) & (NBUF - 1)) == b)
                    def _(b=b):
                        idxb, srcb, rows, sem, scat = bufs[b]
                        pltpu.make_async_copy(xa.at[srcb], rows, sem).wait()
                        pltpu.async_copy(rows, acc.at[idxb], scat, add=True)
        for b in range(NBUF):
            @pl.when(nb >= b + 1)
            def _(b=b):
                idxb, srcb, rows, sem, scat = bufs[b]
                pltpu.make_async_copy(rows, acc.at[idxb], scat).wait()
        pltpu.make_async_copy(cnt_v, cnt_sh.at[ident], souts).wait()
        plsc.subcore_barrier()

        obase = s * (P8 // NSUB)
        for j in range(7):
            pltpu.async_copy(acc.at[pl.ds(obase + j * 56, 56)],
                             outh.at[rid, pl.ds(obase + j * 56, 56)], souts)
        pltpu.async_copy(acc.at[pl.ds(obase + 392, 32)],
                         outh.at[rid, pl.ds(obase + 392, 32)], souts)
        @pl.when(s < CROWS // 8)
        def _():
            pltpu.async_copy(cnt_sh.at[pl.ds(s * 8, 8)],
                             outch.at[rid, pl.ds(s * 8, 8)], souts)
        for j in range(7):
            pltpu.make_async_copy(
                acc.at[pl.ds(obase + j * 56, 56)],
                outh.at[rid, pl.ds(obase + j * 56, 56)], souts).wait()
        pltpu.make_async_copy(acc.at[pl.ds(obase + 392, 32)],
                              outh.at[rid, pl.ds(obase + 392, 32)],
                              souts).wait()
        @pl.when(s < CROWS // 8)
        def _():
            pltpu.make_async_copy(cnt_sh.at[pl.ds(s * 8, 8)],
                                  outch.at[rid, pl.ds(s * 8, 8)],
                                  souts).wait()
        plsc.subcore_barrier()


def _sc_scatter(xa, key, src, identa):
    mesh = plsc.VectorSubcoreMesh(core_axis_name="c", subcore_axis_name="s")
    return pl.kernel(
        _sc_body,
        mesh=mesh,
        compiler_params=pltpu.CompilerParams(needs_layout_passes=False),
        out_type=(
            jax.ShapeDtypeStruct((NRANGE, P8, C), jnp.float32),
            jax.ShapeDtypeStruct((NRANGE, CROWS, C), jnp.float32),
        ),
        scratch_types=[
            pltpu.VMEM((ECH,), jnp.int32),
            pltpu.VMEM((ECH,), jnp.int32),
            pltpu.VMEM((ECH,), jnp.int32),
            pltpu.VMEM((ECH,), jnp.int32),
            pltpu.VMEM((CAP,), jnp.int32),
            pltpu.VMEM((B,), jnp.int32),
            pltpu.VMEM((B,), jnp.int32),
            pltpu.VMEM((B,), jnp.int32),
            pltpu.VMEM((B,), jnp.int32),
            pltpu.VMEM((B,), jnp.int32),
            pltpu.VMEM((B,), jnp.int32),
            pltpu.VMEM((B,), jnp.int32),
            pltpu.VMEM((B,), jnp.int32),
            pltpu.VMEM((B,), jnp.int32),
            pltpu.VMEM((B,), jnp.int32),
            pltpu.VMEM((B,), jnp.int32),
            pltpu.VMEM((B,), jnp.int32),
            pltpu.VMEM((B,), jnp.int32),
            pltpu.VMEM((B,), jnp.int32),
            pltpu.VMEM((B,), jnp.int32),
            pltpu.VMEM((B,), jnp.int32),
            pltpu.VMEM((B, C), jnp.float32),
            pltpu.VMEM((B, C), jnp.float32),
            pltpu.VMEM((B, C), jnp.float32),
            pltpu.VMEM((B, C), jnp.float32),
            pltpu.VMEM((B, C), jnp.float32),
            pltpu.VMEM((B, C), jnp.float32),
            pltpu.VMEM((B, C), jnp.float32),
            pltpu.VMEM((B, C), jnp.float32),
            pltpu.VMEM((ZROWS, C), jnp.float32),
            pltpu.VMEM((CROWS, C), jnp.float32),
            pltpu.VMEM((CROWS,), jnp.int32),
            pltpu.VMEM_SHARED((ACC_ROWS, C), jnp.float32),
            pltpu.VMEM_SHARED((CROWS, C), jnp.float32),
            pltpu.SemaphoreType.DMA,
            pltpu.SemaphoreType.DMA,
            pltpu.SemaphoreType.DMA,
            pltpu.SemaphoreType.DMA,
            pltpu.SemaphoreType.DMA,
            pltpu.SemaphoreType.DMA,
            pltpu.SemaphoreType.DMA,
            pltpu.SemaphoreType.DMA,
            pltpu.SemaphoreType.DMA,
            pltpu.SemaphoreType.DMA,
            pltpu.SemaphoreType.DMA,
            pltpu.SemaphoreType.DMA,
            pltpu.SemaphoreType.DMA,
            pltpu.SemaphoreType.DMA,
            pltpu.SemaphoreType.DMA,
            pltpu.SemaphoreType.DMA,
            pltpu.SemaphoreType.DMA,
            pltpu.SemaphoreType.DMA,
            pltpu.SemaphoreType.DMA,
        ],
    )(xa, key, src, identa)


def _tc_body(x_ref, s_ref, c_ref, w_ref, root_ref, bias_ref, o_ref):
    acc = jnp.dot(x_ref[...], root_ref[...],
                  preferred_element_type=jnp.float32) + bias_ref[...]
    cnt = jnp.maximum(c_ref[...], 1.0)
    for t in range(R):
        mean = s_ref[:, t, :] / cnt[:, t:t + 1]
        acc = acc + jnp.dot(mean, w_ref[t], preferred_element_type=jnp.float32)
    o_ref[...] = jnp.maximum(acc, 0.0)


def _tc_combine(x, s3, cnt2, weight, root, bias2):
    BN = 400
    return pl.pallas_call(
        _tc_body,
        grid=(N // BN,),
        in_specs=[
            pl.BlockSpec((BN, C), lambda i: (i, 0)),
            pl.BlockSpec((BN, R, C), lambda i: (i, 0, 0)),
            pl.BlockSpec((BN, R), lambda i: (i, 0)),
            pl.BlockSpec((R, C, C), lambda i: (0, 0, 0)),
            pl.BlockSpec((C, C), lambda i: (0, 0)),
            pl.BlockSpec((1, C), lambda i: (0, 0)),
        ],
        out_specs=pl.BlockSpec((BN, C), lambda i: (i, 0)),
        out_shape=jax.ShapeDtypeStruct((N, C), jnp.float32),
    )(x, s3, cnt2, weight, root, bias2)


@jax.jit
def kernel(node_features, edge_index, edge_type, weight, root, bias):
    x = node_features.astype(jnp.float32)
    src = edge_index[0].astype(jnp.int32)
    dst = edge_index[1].astype(jnp.int32)
    et = edge_type.astype(jnp.int32)
    key = dst * 8 + et
    identa = jnp.arange(CROWS, dtype=jnp.int32)
    sums, cnts = _sc_scatter(x, key, src, identa)
    s3 = sums.reshape(NRANGE * P, R, C)
    cnt2 = cnts.reshape(NRANGE, CROWS * C)[:, :P8].reshape(NRANGE * P, R)
    return _tc_combine(x, s3, cnt2, weight, root.astype(jnp.float32),
                       bias.reshape(1, C).astype(jnp.float32))
